# baseline (speedup 1.0000x reference)
import functools

import jax
import jax.numpy as jnp
from jax import lax
from jax.experimental import pallas as pl
from jax.experimental.pallas import tpu as pltpu
from jax.experimental.pallas import tpu_sc as plsc

ML = 8
F = 128
PAD = 112

NL = 10000
NA = 4096
E = 32768
S = ML * NA
NLp = NL + PAD
NC = 2
NS = 16
NW = NC * NS
EPW = E // NW
SPW = S // NW
RPS = NLp // NS

_SC_MESH = plsc.VectorSubcoreMesh(core_axis_name="c", subcore_axis_name="s")
_SC_PARAMS = pltpu.CompilerParams(needs_layout_passes=False)


def _ds(base, size):
    return pl.ds(pl.multiple_of(base, 8), size)


def _inv_body(pid_hbm, seq_hbm, lid_hbm, src_hbm, flat_hbm, cap_hbm,
              pid_v, seq_v, lid_v, src_v, flat_v, cap_v):
    w = lax.axis_index("s") * NC + lax.axis_index("c")
    lo = w * SPW
    pltpu.sync_copy(pid_hbm, pid_v)
    pltpu.sync_copy(seq_hbm, seq_v)
    pltpu.sync_copy(lid_hbm, lid_v)

    def init_body(j, carry):
        spread = (lax.iota(jnp.int32, 16) + j * 16) & 63
        src_v[pl.ds(j * 16, 16)] = NL + spread
        return carry

    lax.fori_loop(0, SPW // 16, init_body, 0)

    def scan_body(i, carry):
        sq = seq_v[pl.ds(i * 16, 16)]
        f = sq * NA + pid_v[pl.ds(i * 16, 16)]
        l = lid_v[pl.ds(i * 16, 16)]
        fl = f - lo
        m = (fl >= 0) & (fl < SPW)
        plsc.store_scatter(src_v, [fl], l, mask=m)
        return carry

    lax.fori_loop(0, E // 16, scan_body, 0)
    pltpu.sync_copy(src_v, src_hbm.at[_ds(lo, SPW)])

    ebase = w * EPW

    def flat_body(k, mx):
        sq = seq_v[pl.ds(ebase + k * 16, 16)]
        f = sq * NA + pid_v[pl.ds(ebase + k * 16, 16)]
        flat_v[pl.ds(k * 16, 16)] = f
        return jnp.maximum(mx, sq)

    mx = lax.fori_loop(0, EPW // 16, flat_body, jnp.zeros((16,), jnp.int32))
    pltpu.sync_copy(flat_v, flat_hbm.at[_ds(ebase, EPW)])
    for r in range(8):
        for j in range(8):
            cap_v[r, pl.ds(j * 16, 16)] = mx
    pltpu.sync_copy(cap_v, cap_hbm.at[_ds(w * 8, 8)])


_sc_invert = functools.partial(
    pl.kernel, _inv_body, mesh=_SC_MESH, compiler_params=_SC_PARAMS,
    out_type=(jax.ShapeDtypeStruct((S,), jnp.int32),
              jax.ShapeDtypeStruct((E,), jnp.int32),
              jax.ShapeDtypeStruct((NW * 8, 128), jnp.int32)),
    scratch_types=[pltpu.VMEM((E,), jnp.int32),
                   pltpu.VMEM((E,), jnp.int32),
                   pltpu.VMEM((E,), jnp.int32),
                   pltpu.VMEM((SPW,), jnp.int32),
                   pltpu.VMEM((EPW,), jnp.int32),
                   pltpu.VMEM((8, 128), jnp.int32)],
)()


def _gat_body(ls_hbm, src2_hbm, mi_hbm, idx_v, rows_v, sem):
    w = lax.axis_index("s") * NC + lax.axis_index("c")
    pltpu.sync_copy(src2_hbm.at[_ds(w * 8, 8)], idx_v)
    for cc in range(8):
        pltpu.async_copy(ls_hbm.at[idx_v.at[cc]], rows_v, sem).wait()
        pltpu.sync_copy(rows_v, mi_hbm.at[_ds(w * SPW + cc * 128, 128)])


_sc_gather = functools.partial(
    pl.kernel, _gat_body, mesh=_SC_MESH, compiler_params=_SC_PARAMS,
    out_type=jax.ShapeDtypeStruct((S, F), jnp.float32),
    scratch_types=[pltpu.VMEM((8, 128), jnp.int32),
                   pltpu.VMEM((128, F), jnp.float32),
                   pltpu.SemaphoreType.DMA],
)()


def _agg_body(mseq_hbm, flat2_hbm, lid2_hbm, zeros_hbm, parts_hbm,
              fidx, lidx, rows_v, sp, sem):
    c = lax.axis_index("c")
    s = lax.axis_index("s")
    w = s * NC + c
    pltpu.sync_copy(zeros_hbm.at[_ds(s * RPS, RPS)],
                    sp.at[_ds(s * RPS, RPS)])
    pltpu.sync_copy(flat2_hbm.at[_ds(w * 8, 8)], fidx)
    pltpu.sync_copy(lid2_hbm.at[_ds(w * 8, 8)], lidx)
    plsc.subcore_barrier()
    for cc in range(8):
        pltpu.async_copy(mseq_hbm.at[fidx.at[cc]], rows_v, sem).wait()
        pltpu.sync_copy(rows_v, sp.at[lidx.at[cc]], add=True)
    plsc.subcore_barrier()
    pltpu.sync_copy(sp.at[_ds(s * RPS, RPS)],
                    parts_hbm.at[c, _ds(s * RPS, RPS)])


_sc_agg = functools.partial(
    pl.kernel, _agg_body, mesh=_SC_MESH, compiler_params=_SC_PARAMS,
    out_type=jax.ShapeDtypeStruct((NC, NLp, F), jnp.float32),
    scratch_types=[pltpu.VMEM((8, 128), jnp.int32),
                   pltpu.VMEM((8, 128), jnp.int32),
                   pltpu.VMEM((128, F), jnp.float32),
                   pltpu.VMEM_SHARED((NLp, F), jnp.float32),
                   pltpu.SemaphoreType.DMA],
)()


def _gru_math(x, h, gi_w, gh_w, b_ih, b_hh):
    gi = jax.lax.dot_general(x, gi_w, (((1,), (0,)), ((), ())),
                             preferred_element_type=jnp.float32) + b_ih
    gh = jax.lax.dot_general(h, gh_w, (((1,), (0,)), ((), ())),
                             preferred_element_type=jnp.float32) + b_hh
    r = jax.nn.sigmoid(gi[:, :F] + gh[:, :F])
    z = jax.nn.sigmoid(gi[:, F:2 * F] + gh[:, F:2 * F])
    n = jnp.tanh(gi[:, 2 * F:] + r * gh[:, 2 * F:])
    return (1.0 - z) * n + z * h


def _scan_body(cap_ref, mi_ref, ps_ref, wih_ref, whh_ref, bih_ref, bhh_ref,
               out_ref, h_ref):
    t = pl.program_id(0)

    @pl.when(t == 0)
    def _():
        h_ref[...] = ps_ref[...]

    x = mi_ref[0]
    h = h_ref[...]
    h_new = _gru_math(x, h, wih_ref[...], whh_ref[...], bih_ref[...],
                      bhh_ref[...])
    keep = t < cap_ref[0]
    h2 = jnp.where(keep, h_new, h)
    h_ref[...] = h2
    out_ref[0] = h2


def _msg_scan(mi, ps, wih_t, whh_t, bih, bhh, cap):
    return pl.pallas_call(
        _scan_body,
        grid_spec=pltpu.PrefetchScalarGridSpec(
            num_scalar_prefetch=1,
            grid=(ML,),
            in_specs=[
                pl.BlockSpec((1, NA, F), lambda t, *_: (t, 0, 0)),
                pl.BlockSpec((NA, F), lambda t, *_: (0, 0)),
                pl.BlockSpec((F, 3 * F), lambda t, *_: (0, 0)),
                pl.BlockSpec((F, 3 * F), lambda t, *_: (0, 0)),
                pl.BlockSpec((1, 3 * F), lambda t, *_: (0, 0)),
                pl.BlockSpec((1, 3 * F), lambda t, *_: (0, 0)),
            ],
            out_specs=pl.BlockSpec((1, NA, F), lambda t, *_: (t, 0, 0)),
            scratch_shapes=[pltpu.VMEM((NA, F), jnp.float32)],
        ),
        out_shape=jax.ShapeDtypeStruct((ML, NA, F), jnp.float32),
    )(cap, mi, ps, wih_t, whh_t, bih, bhh)


def _update_body(nl_ref, x_ref, h_ref, wih_ref, whh_ref, bih_ref, bhh_ref,
                 out_ref):
    i = pl.program_id(0)
    rows = h_ref.shape[0]
    x = x_ref[0] + x_ref[1]
    h = h_ref[...]
    h_new = _gru_math(x, h, wih_ref[...], whh_ref[...], bih_ref[...],
                      bhh_ref[...])
    row = jax.lax.broadcasted_iota(jnp.int32, (rows, 1), 0) + i * rows
    out_ref[...] = jnp.where(row < nl_ref[0], h_new, 0.0)


def _link_update(parts, ls_ext, wih_t, whh_t, bih, bhh, nl):
    blk = NLp // 8
    return pl.pallas_call(
        _update_body,
        grid_spec=pltpu.PrefetchScalarGridSpec(
            num_scalar_prefetch=1,
            grid=(8,),
            in_specs=[
                pl.BlockSpec((NC, blk, F), lambda i, *_: (0, i, 0)),
                pl.BlockSpec((blk, F), lambda i, *_: (i, 0)),
                pl.BlockSpec((F, 3 * F), lambda i, *_: (0, 0)),
                pl.BlockSpec((F, 3 * F), lambda i, *_: (0, 0)),
                pl.BlockSpec((1, 3 * F), lambda i, *_: (0, 0)),
                pl.BlockSpec((1, 3 * F), lambda i, *_: (0, 0)),
            ],
            out_specs=pl.BlockSpec((blk, F), lambda i, *_: (i, 0)),
            scratch_shapes=[],
        ),
        out_shape=jax.ShapeDtypeStruct((NLp, F), jnp.float32),
    )(nl, parts, ls_ext, wih_t, whh_t, bih, bhh)


def _selu(x):
    alpha = 1.6732632423543772848170429916717
    scale = 1.0507009873554804934193349852946
    return scale * jnp.where(x > 0, x, alpha * (jnp.exp(x) - 1.0))


def _readout_body(ps_ref, w1_ref, b1_ref, w2_ref, b2_ref, wo_ref, bo_ref,
                  out_ref):
    h = _selu(jax.lax.dot_general(ps_ref[...], w1_ref[...],
                                  (((1,), (0,)), ((), ())),
                                  preferred_element_type=jnp.float32)
              + b1_ref[...])
    h = _selu(jax.lax.dot_general(h, w2_ref[...], (((1,), (0,)), ((), ())),
                                  preferred_element_type=jnp.float32)
              + b2_ref[...])
    out_ref[...] = jax.lax.dot_general(h, wo_ref[...],
                                       (((1,), (0,)), ((), ())),
                                       preferred_element_type=jnp.float32) \
        + bo_ref[...]


def _readout(ps, w1_t, b1, w2_t, b2, wo_t, bo):
    return pl.pallas_call(
        _readout_body,
        out_shape=jax.ShapeDtypeStruct((NA, F), jnp.float32),
    )(ps, w1_t, b1, w2_t, b2, wo_t, bo)


def _pad_body(ls_ref, out_ref):
    nl = ls_ref.shape[0]
    out_ref[:nl, :] = ls_ref[...]
    out_ref[nl:, :] = jnp.zeros_like(out_ref[nl:, :])


def _pad_links(ls):
    return pl.pallas_call(
        _pad_body,
        out_shape=jax.ShapeDtypeStruct((NLp, F), jnp.float32),
    )(ls)


def kernel(link_state, path_state, link_id, path_id, sequence, num_actions,
           W_ih_m, W_hh_m, b_ih_m, b_hh_m, W_ih_u, W_hh_u, b_ih_u, b_hh_u,
           W_r1, b_r1, W_r2, b_r2, W_out, b_out):
    T = 4

    lid = link_id.astype(jnp.int32)
    pid = path_id.astype(jnp.int32)
    seq = sequence.astype(jnp.int32)

    wm_ih = W_ih_m.T
    wm_hh = W_hh_m.T
    wu_ih = W_ih_u.T
    wu_hh = W_hh_u.T
    bm_ih = b_ih_m.reshape(1, -1)
    bm_hh = b_hh_m.reshape(1, -1)
    bu_ih = b_ih_u.reshape(1, -1)
    bu_hh = b_hh_u.reshape(1, -1)
    w1_t = W_r1.T
    w2_t = W_r2.T
    wo_t = jnp.pad(W_out.T, ((0, 0), (0, F - W_out.shape[0])))
    b1 = b_r1.reshape(1, -1)
    b2 = b_r2.reshape(1, -1)
    bo = jnp.pad(b_out.reshape(1, -1), ((0, 0), (0, F - b_out.shape[0])))

    src, flat, capm = _sc_invert(pid, seq, lid)
    src2 = src.reshape(S // 128, 128)
    flat2 = flat.reshape(E // 128, 128)
    lid2 = lid.reshape(E // 128, 128)
    cap = jnp.minimum(jnp.max(capm) + 1, num_actions).astype(jnp.int32)
    cap_arr = cap.reshape(1)
    nl_arr = jnp.full((1,), NL, jnp.int32)
    zeros_hbm = jnp.zeros((NLp, F), jnp.float32)

    ls_ext = _pad_links(link_state)
    ps = path_state
    for r in range(T):
        mi = _sc_gather(ls_ext, src2).reshape(ML, NA, F)
        m_seq = _msg_scan(mi, ps, wm_ih, wm_hh, bm_ih, bm_hh, cap_arr)
        ps = m_seq[ML - 1]
        if r < T - 1:
            parts = _sc_agg(m_seq.reshape(S, F), flat2, lid2, zeros_hbm)
            ls_ext = _link_update(parts, ls_ext, wu_ih, wu_hh, bu_ih, bu_hh,
                                  nl_arr)
    out = _readout(ps, w1_t, b1, w2_t, b2, wo_t, bo)
    return out[:, :1]


# speedup vs baseline: 1.0323x; 1.0323x over previous
import functools

import jax
import jax.numpy as jnp
from jax import lax
from jax.experimental import pallas as pl
from jax.experimental.pallas import tpu as pltpu
from jax.experimental.pallas import tpu_sc as plsc

ML = 8
F = 128
PAD = 112

NL = 10000
NA = 4096
E = 32768
S = ML * NA
NLp = NL + PAD
NC = 2
NS = 16
NW = NC * NS
EPW = E // NW
SPW = S // NW
RPS = NLp // NS

_SC_MESH = plsc.VectorSubcoreMesh(core_axis_name="c", subcore_axis_name="s")
_SC_PARAMS = pltpu.CompilerParams(needs_layout_passes=False)


def _ds(base, size):
    return pl.ds(pl.multiple_of(base, 8), size)


def _inv_body(pid_hbm, seq_hbm, lid_hbm, src_hbm, flat_hbm, cap_hbm,
              pid_v, seq_v, lid_v, src_v, flat_v, cap_v):
    w = lax.axis_index("s") * NC + lax.axis_index("c")
    lo = w * SPW
    pltpu.sync_copy(pid_hbm, pid_v)
    pltpu.sync_copy(seq_hbm, seq_v)
    pltpu.sync_copy(lid_hbm, lid_v)

    def init_body(j, carry):
        spread = (lax.iota(jnp.int32, 16) + j * 16) & 63
        src_v[pl.ds(j * 16, 16)] = NL + spread
        return carry

    lax.fori_loop(0, SPW // 16, init_body, 0)

    def scan_body(i, carry):
        sq = seq_v[pl.ds(i * 16, 16)]
        f = sq * NA + pid_v[pl.ds(i * 16, 16)]
        l = lid_v[pl.ds(i * 16, 16)]
        fl = f - lo
        m = (fl >= 0) & (fl < SPW)
        plsc.store_scatter(src_v, [fl], l, mask=m)
        return carry

    lax.fori_loop(0, E // 16, scan_body, 0)
    pltpu.sync_copy(src_v, src_hbm.at[_ds(lo, SPW)])

    ebase = w * EPW

    def flat_body(k, mx):
        sq = seq_v[pl.ds(ebase + k * 16, 16)]
        f = sq * NA + pid_v[pl.ds(ebase + k * 16, 16)]
        flat_v[pl.ds(k * 16, 16)] = f
        return jnp.maximum(mx, sq)

    mx = lax.fori_loop(0, EPW // 16, flat_body, jnp.zeros((16,), jnp.int32))
    pltpu.sync_copy(flat_v, flat_hbm.at[_ds(ebase, EPW)])
    for r in range(8):
        for j in range(8):
            cap_v[r, pl.ds(j * 16, 16)] = mx
    pltpu.sync_copy(cap_v, cap_hbm.at[_ds(w * 8, 8)])


_sc_invert = functools.partial(
    pl.kernel, _inv_body, mesh=_SC_MESH, compiler_params=_SC_PARAMS,
    out_type=(jax.ShapeDtypeStruct((S,), jnp.int32),
              jax.ShapeDtypeStruct((E,), jnp.int32),
              jax.ShapeDtypeStruct((NW * 8, 128), jnp.int32)),
    scratch_types=[pltpu.VMEM((E,), jnp.int32),
                   pltpu.VMEM((E,), jnp.int32),
                   pltpu.VMEM((E,), jnp.int32),
                   pltpu.VMEM((SPW,), jnp.int32),
                   pltpu.VMEM((EPW,), jnp.int32),
                   pltpu.VMEM((8, 128), jnp.int32)],
)()


def _gat_body(ls_hbm, src2_hbm, mi_hbm, idx_v, rows_a, rows_b,
              sg_a, sg_b, sw_a, sw_b):
    w = lax.axis_index("s") * NC + lax.axis_index("c")
    pltpu.sync_copy(src2_hbm.at[_ds(w * 8, 8)], idx_v)
    rows = (rows_a, rows_b)
    sg = (sg_a, sg_b)
    sw = (sw_a, sw_b)
    gd = [None, None]
    wd = [None, None]
    gd[0] = pltpu.async_copy(ls_hbm.at[idx_v.at[0]], rows_a, sg_a)
    for cc in range(8):
        b = cc & 1
        gd[b].wait()
        if cc + 1 < 8:
            if cc >= 1:
                wd[1 - b].wait()
            gd[1 - b] = pltpu.async_copy(ls_hbm.at[idx_v.at[cc + 1]],
                                         rows[1 - b], sg[1 - b])
        wd[b] = pltpu.async_copy(rows[b],
                                 mi_hbm.at[_ds(w * SPW + cc * 128, 128)],
                                 sw[b])
    wd[0].wait()
    wd[1].wait()


_sc_gather = functools.partial(
    pl.kernel, _gat_body, mesh=_SC_MESH, compiler_params=_SC_PARAMS,
    out_type=jax.ShapeDtypeStruct((S, F), jnp.float32),
    scratch_types=[pltpu.VMEM((8, 128), jnp.int32),
                   pltpu.VMEM((128, F), jnp.float32),
                   pltpu.VMEM((128, F), jnp.float32),
                   pltpu.SemaphoreType.DMA, pltpu.SemaphoreType.DMA,
                   pltpu.SemaphoreType.DMA, pltpu.SemaphoreType.DMA],
)()


def _agg_body(mseq_hbm, flat2_hbm, lid2_hbm, zeros_hbm, parts_hbm,
              fidx, lidx, rows_a, rows_b, sp, sg_a, sg_b, sw_a, sw_b):
    c = lax.axis_index("c")
    s = lax.axis_index("s")
    w = s * NC + c
    pltpu.sync_copy(zeros_hbm.at[_ds(s * RPS, RPS)],
                    sp.at[_ds(s * RPS, RPS)])
    pltpu.sync_copy(flat2_hbm.at[_ds(w * 8, 8)], fidx)
    pltpu.sync_copy(lid2_hbm.at[_ds(w * 8, 8)], lidx)
    plsc.subcore_barrier()
    rows = (rows_a, rows_b)
    sg = (sg_a, sg_b)
    sw = (sw_a, sw_b)
    gd = [None, None]
    wd = [None, None]
    gd[0] = pltpu.async_copy(mseq_hbm.at[fidx.at[0]], rows_a, sg_a)
    for cc in range(8):
        b = cc & 1
        gd[b].wait()
        if cc + 1 < 8:
            if cc >= 1:
                wd[1 - b].wait()
            gd[1 - b] = pltpu.async_copy(mseq_hbm.at[fidx.at[cc + 1]],
                                         rows[1 - b], sg[1 - b])
        wd[b] = pltpu.async_copy(rows[b], sp.at[lidx.at[cc]], sw[b],
                                 add=True)
    wd[0].wait()
    wd[1].wait()
    plsc.subcore_barrier()
    pltpu.sync_copy(sp.at[_ds(s * RPS, RPS)],
                    parts_hbm.at[c, _ds(s * RPS, RPS)])


_sc_agg = functools.partial(
    pl.kernel, _agg_body, mesh=_SC_MESH, compiler_params=_SC_PARAMS,
    out_type=jax.ShapeDtypeStruct((NC, NLp, F), jnp.float32),
    scratch_types=[pltpu.VMEM((8, 128), jnp.int32),
                   pltpu.VMEM((8, 128), jnp.int32),
                   pltpu.VMEM((128, F), jnp.float32),
                   pltpu.VMEM((128, F), jnp.float32),
                   pltpu.VMEM_SHARED((NLp, F), jnp.float32),
                   pltpu.SemaphoreType.DMA, pltpu.SemaphoreType.DMA,
                   pltpu.SemaphoreType.DMA, pltpu.SemaphoreType.DMA],
)()


def _gru_math(x, h, gi_w, gh_w, b_ih, b_hh):
    gi = jax.lax.dot_general(x, gi_w, (((1,), (0,)), ((), ())),
                             preferred_element_type=jnp.float32) + b_ih
    gh = jax.lax.dot_general(h, gh_w, (((1,), (0,)), ((), ())),
                             preferred_element_type=jnp.float32) + b_hh
    r = jax.nn.sigmoid(gi[:, :F] + gh[:, :F])
    z = jax.nn.sigmoid(gi[:, F:2 * F] + gh[:, F:2 * F])
    n = jnp.tanh(gi[:, 2 * F:] + r * gh[:, 2 * F:])
    return (1.0 - z) * n + z * h


def _scan_body(cap_ref, mi_ref, ps_ref, wih_ref, whh_ref, bih_ref, bhh_ref,
               out_ref, h_ref):
    t = pl.program_id(0)

    @pl.when(t == 0)
    def _():
        h_ref[...] = ps_ref[...]

    x = mi_ref[0]
    h = h_ref[...]
    h_new = _gru_math(x, h, wih_ref[...], whh_ref[...], bih_ref[...],
                      bhh_ref[...])
    keep = t < cap_ref[0]
    h2 = jnp.where(keep, h_new, h)
    h_ref[...] = h2
    out_ref[0] = h2


def _msg_scan(mi, ps, wih_t, whh_t, bih, bhh, cap):
    return pl.pallas_call(
        _scan_body,
        grid_spec=pltpu.PrefetchScalarGridSpec(
            num_scalar_prefetch=1,
            grid=(ML,),
            in_specs=[
                pl.BlockSpec((1, NA, F), lambda t, *_: (t, 0, 0)),
                pl.BlockSpec((NA, F), lambda t, *_: (0, 0)),
                pl.BlockSpec((F, 3 * F), lambda t, *_: (0, 0)),
                pl.BlockSpec((F, 3 * F), lambda t, *_: (0, 0)),
                pl.BlockSpec((1, 3 * F), lambda t, *_: (0, 0)),
                pl.BlockSpec((1, 3 * F), lambda t, *_: (0, 0)),
            ],
            out_specs=pl.BlockSpec((1, NA, F), lambda t, *_: (t, 0, 0)),
            scratch_shapes=[pltpu.VMEM((NA, F), jnp.float32)],
        ),
        out_shape=jax.ShapeDtypeStruct((ML, NA, F), jnp.float32),
    )(cap, mi, ps, wih_t, whh_t, bih, bhh)


def _update_body(nl_ref, x_ref, h_ref, wih_ref, whh_ref, bih_ref, bhh_ref,
                 out_ref):
    i = pl.program_id(0)
    rows = h_ref.shape[0]
    x = x_ref[0] + x_ref[1]
    h = h_ref[...]
    h_new = _gru_math(x, h, wih_ref[...], whh_ref[...], bih_ref[...],
                      bhh_ref[...])
    row = jax.lax.broadcasted_iota(jnp.int32, (rows, 1), 0) + i * rows
    out_ref[...] = jnp.where(row < nl_ref[0], h_new, 0.0)


def _link_update(parts, ls_ext, wih_t, whh_t, bih, bhh, nl):
    blk = NLp // 8
    return pl.pallas_call(
        _update_body,
        grid_spec=pltpu.PrefetchScalarGridSpec(
            num_scalar_prefetch=1,
            grid=(8,),
            in_specs=[
                pl.BlockSpec((NC, blk, F), lambda i, *_: (0, i, 0)),
                pl.BlockSpec((blk, F), lambda i, *_: (i, 0)),
                pl.BlockSpec((F, 3 * F), lambda i, *_: (0, 0)),
                pl.BlockSpec((F, 3 * F), lambda i, *_: (0, 0)),
                pl.BlockSpec((1, 3 * F), lambda i, *_: (0, 0)),
                pl.BlockSpec((1, 3 * F), lambda i, *_: (0, 0)),
            ],
            out_specs=pl.BlockSpec((blk, F), lambda i, *_: (i, 0)),
            scratch_shapes=[],
        ),
        out_shape=jax.ShapeDtypeStruct((NLp, F), jnp.float32),
    )(nl, parts, ls_ext, wih_t, whh_t, bih, bhh)


def _selu(x):
    alpha = 1.6732632423543772848170429916717
    scale = 1.0507009873554804934193349852946
    return scale * jnp.where(x > 0, x, alpha * (jnp.exp(x) - 1.0))


def _readout_body(ps_ref, w1_ref, b1_ref, w2_ref, b2_ref, wo_ref, bo_ref,
                  out_ref):
    h = _selu(jax.lax.dot_general(ps_ref[...], w1_ref[...],
                                  (((1,), (0,)), ((), ())),
                                  preferred_element_type=jnp.float32)
              + b1_ref[...])
    h = _selu(jax.lax.dot_general(h, w2_ref[...], (((1,), (0,)), ((), ())),
                                  preferred_element_type=jnp.float32)
              + b2_ref[...])
    out_ref[...] = jax.lax.dot_general(h, wo_ref[...],
                                       (((1,), (0,)), ((), ())),
                                       preferred_element_type=jnp.float32) \
        + bo_ref[...]


def _readout(ps, w1_t, b1, w2_t, b2, wo_t, bo):
    return pl.pallas_call(
        _readout_body,
        out_shape=jax.ShapeDtypeStruct((NA, F), jnp.float32),
    )(ps, w1_t, b1, w2_t, b2, wo_t, bo)


def _pad_body(ls_ref, out_ref):
    nl = ls_ref.shape[0]
    out_ref[:nl, :] = ls_ref[...]
    out_ref[nl:, :] = jnp.zeros_like(out_ref[nl:, :])


def _pad_links(ls):
    return pl.pallas_call(
        _pad_body,
        out_shape=jax.ShapeDtypeStruct((NLp, F), jnp.float32),
    )(ls)


def kernel(link_state, path_state, link_id, path_id, sequence, num_actions,
           W_ih_m, W_hh_m, b_ih_m, b_hh_m, W_ih_u, W_hh_u, b_ih_u, b_hh_u,
           W_r1, b_r1, W_r2, b_r2, W_out, b_out):
    T = 4

    lid = link_id.astype(jnp.int32)
    pid = path_id.astype(jnp.int32)
    seq = sequence.astype(jnp.int32)

    wm_ih = W_ih_m.T
    wm_hh = W_hh_m.T
    wu_ih = W_ih_u.T
    wu_hh = W_hh_u.T
    bm_ih = b_ih_m.reshape(1, -1)
    bm_hh = b_hh_m.reshape(1, -1)
    bu_ih = b_ih_u.reshape(1, -1)
    bu_hh = b_hh_u.reshape(1, -1)
    w1_t = W_r1.T
    w2_t = W_r2.T
    wo_t = jnp.pad(W_out.T, ((0, 0), (0, F - W_out.shape[0])))
    b1 = b_r1.reshape(1, -1)
    b2 = b_r2.reshape(1, -1)
    bo = jnp.pad(b_out.reshape(1, -1), ((0, 0), (0, F - b_out.shape[0])))

    src, flat, capm = _sc_invert(pid, seq, lid)
    src2 = src.reshape(S // 128, 128)
    flat2 = flat.reshape(E // 128, 128)
    lid2 = lid.reshape(E // 128, 128)
    cap = jnp.minimum(jnp.max(capm) + 1, num_actions).astype(jnp.int32)
    cap_arr = cap.reshape(1)
    nl_arr = jnp.full((1,), NL, jnp.int32)
    zeros_hbm = jnp.zeros((NLp, F), jnp.float32)

    ls_ext = _pad_links(link_state)
    ps = path_state
    for r in range(T):
        mi = _sc_gather(ls_ext, src2).reshape(ML, NA, F)
        m_seq = _msg_scan(mi, ps, wm_ih, wm_hh, bm_ih, bm_hh, cap_arr)
        ps = m_seq[ML - 1]
        if r < T - 1:
            parts = _sc_agg(m_seq.reshape(S, F), flat2, lid2, zeros_hbm)
            ls_ext = _link_update(parts, ls_ext, wu_ih, wu_hh, bu_ih, bu_hh,
                                  nl_arr)
    out = _readout(ps, w1_t, b1, w2_t, b2, wo_t, bo)
    return out[:, :1]


# speedup vs baseline: 1.0450x; 1.0123x over previous
import functools

import jax
import jax.numpy as jnp
from jax import lax
from jax.experimental import pallas as pl
from jax.experimental.pallas import tpu as pltpu
from jax.experimental.pallas import tpu_sc as plsc

ML = 8
F = 128
PAD = 112

NL = 10000
NA = 4096
E = 32768
S = ML * NA
NLp = NL + PAD
NC = 2
NS = 16
NW = NC * NS
EPW = E // NW
SPW = S // NW
RPS = NLp // NS

_SC_MESH = plsc.VectorSubcoreMesh(core_axis_name="c", subcore_axis_name="s")
_SC_PARAMS = pltpu.CompilerParams(needs_layout_passes=False)


def _ds(base, size):
    return pl.ds(pl.multiple_of(base, 8), size)


def _inv_body(pid_hbm, seq_hbm, lid_hbm, src_hbm, flat_hbm, cap_hbm,
              pid_v, seq_v, lid_v, src_v, flat_v, cap_v):
    w = lax.axis_index("s") * NC + lax.axis_index("c")
    lo = w * SPW
    pltpu.sync_copy(pid_hbm, pid_v)
    pltpu.sync_copy(seq_hbm, seq_v)
    pltpu.sync_copy(lid_hbm, lid_v)

    def init_body(j, carry):
        spread = (lax.iota(jnp.int32, 16) + j * 16) & 63
        src_v[pl.ds(j * 16, 16)] = NL + spread
        return carry

    lax.fori_loop(0, SPW // 16, init_body, 0)

    def scan_body(i, carry):
        sq = seq_v[pl.ds(i * 16, 16)]
        f = sq * NA + pid_v[pl.ds(i * 16, 16)]
        l = lid_v[pl.ds(i * 16, 16)]
        fl = f - lo
        m = (fl >= 0) & (fl < SPW)
        plsc.store_scatter(src_v, [fl], l, mask=m)
        return carry

    lax.fori_loop(0, E // 16, scan_body, 0)
    pltpu.sync_copy(src_v, src_hbm.at[_ds(lo, SPW)])

    ebase = w * EPW

    def flat_body(k, mx):
        sq = seq_v[pl.ds(ebase + k * 16, 16)]
        f = sq * NA + pid_v[pl.ds(ebase + k * 16, 16)]
        flat_v[pl.ds(k * 16, 16)] = f
        return jnp.maximum(mx, sq)

    mx = lax.fori_loop(0, EPW // 16, flat_body, jnp.zeros((16,), jnp.int32))
    pltpu.sync_copy(flat_v, flat_hbm.at[_ds(ebase, EPW)])
    for r in range(8):
        for j in range(8):
            cap_v[r, pl.ds(j * 16, 16)] = mx
    pltpu.sync_copy(cap_v, cap_hbm.at[_ds(w * 8, 8)])


_sc_invert = functools.partial(
    pl.kernel, _inv_body, mesh=_SC_MESH, compiler_params=_SC_PARAMS,
    out_type=(jax.ShapeDtypeStruct((S,), jnp.int32),
              jax.ShapeDtypeStruct((E,), jnp.int32),
              jax.ShapeDtypeStruct((NW * 8, 128), jnp.int32)),
    scratch_types=[pltpu.VMEM((E,), jnp.int32),
                   pltpu.VMEM((E,), jnp.int32),
                   pltpu.VMEM((E,), jnp.int32),
                   pltpu.VMEM((SPW,), jnp.int32),
                   pltpu.VMEM((EPW,), jnp.int32),
                   pltpu.VMEM((8, 128), jnp.int32)],
)()


def _gat_body(ls_hbm, src2_hbm, mi_hbm, idx_v, rows_a, rows_b,
              sg_a, sg_b, sw_a, sw_b):
    w = lax.axis_index("s") * NC + lax.axis_index("c")
    pltpu.sync_copy(src2_hbm.at[_ds(w * 8, 8)], idx_v)
    rows = (rows_a, rows_b)
    sg = (sg_a, sg_b)
    sw = (sw_a, sw_b)
    gd = [None, None]
    wd = [None, None]
    gd[0] = pltpu.async_copy(ls_hbm.at[idx_v.at[0]], rows_a, sg_a)
    for cc in range(8):
        b = cc & 1
        gd[b].wait()
        if cc + 1 < 8:
            if cc >= 1:
                wd[1 - b].wait()
            gd[1 - b] = pltpu.async_copy(ls_hbm.at[idx_v.at[cc + 1]],
                                         rows[1 - b], sg[1 - b])
        wd[b] = pltpu.async_copy(rows[b],
                                 mi_hbm.at[_ds(w * SPW + cc * 128, 128)],
                                 sw[b])
    wd[0].wait()
    wd[1].wait()


_sc_gather = functools.partial(
    pl.kernel, _gat_body, mesh=_SC_MESH, compiler_params=_SC_PARAMS,
    out_type=jax.ShapeDtypeStruct((S, F), jnp.float32),
    scratch_types=[pltpu.VMEM((8, 128), jnp.int32),
                   pltpu.VMEM((128, F), jnp.float32),
                   pltpu.VMEM((128, F), jnp.float32),
                   pltpu.SemaphoreType.DMA, pltpu.SemaphoreType.DMA,
                   pltpu.SemaphoreType.DMA, pltpu.SemaphoreType.DMA],
)()


def _agg_body(mseq_hbm, flat2_hbm, lid2_hbm, zeros_hbm, parts_hbm,
              fidx, lidx, rows_a, rows_b, sp, sg_a, sg_b, sw_a, sw_b):
    c = lax.axis_index("c")
    s = lax.axis_index("s")
    w = s * NC + c
    pltpu.sync_copy(zeros_hbm.at[_ds(s * RPS, RPS)],
                    sp.at[_ds(s * RPS, RPS)])
    pltpu.sync_copy(flat2_hbm.at[_ds(w * 8, 8)], fidx)
    pltpu.sync_copy(lid2_hbm.at[_ds(w * 8, 8)], lidx)
    plsc.subcore_barrier()
    rows = (rows_a, rows_b)
    sg = (sg_a, sg_b)
    sw = (sw_a, sw_b)
    gd = [None, None]
    wd = [None, None]
    gd[0] = pltpu.async_copy(mseq_hbm.at[fidx.at[0]], rows_a, sg_a)
    for cc in range(8):
        b = cc & 1
        gd[b].wait()
        if cc + 1 < 8:
            if cc >= 1:
                wd[1 - b].wait()
            gd[1 - b] = pltpu.async_copy(mseq_hbm.at[fidx.at[cc + 1]],
                                         rows[1 - b], sg[1 - b])
        wd[b] = pltpu.async_copy(rows[b], sp.at[lidx.at[cc]], sw[b],
                                 add=True)
    wd[0].wait()
    wd[1].wait()
    plsc.subcore_barrier()
    pltpu.sync_copy(sp.at[_ds(s * RPS, RPS)],
                    parts_hbm.at[c, _ds(s * RPS, RPS)])


_sc_agg = functools.partial(
    pl.kernel, _agg_body, mesh=_SC_MESH, compiler_params=_SC_PARAMS,
    out_type=jax.ShapeDtypeStruct((NC, NLp, F), jnp.float32),
    scratch_types=[pltpu.VMEM((8, 128), jnp.int32),
                   pltpu.VMEM((8, 128), jnp.int32),
                   pltpu.VMEM((128, F), jnp.float32),
                   pltpu.VMEM((128, F), jnp.float32),
                   pltpu.VMEM_SHARED((NLp, F), jnp.float32),
                   pltpu.SemaphoreType.DMA, pltpu.SemaphoreType.DMA,
                   pltpu.SemaphoreType.DMA, pltpu.SemaphoreType.DMA],
)()


def _gru_math(x, h, gi_w, gh_w, b_ih, b_hh):
    gi = jax.lax.dot_general(x, gi_w, (((1,), (0,)), ((), ())),
                             preferred_element_type=jnp.float32) + b_ih
    gh = jax.lax.dot_general(h, gh_w, (((1,), (0,)), ((), ())),
                             preferred_element_type=jnp.float32) + b_hh
    r = jax.nn.sigmoid(gi[:, :F] + gh[:, :F])
    z = jax.nn.sigmoid(gi[:, F:2 * F] + gh[:, F:2 * F])
    n = jnp.tanh(gi[:, 2 * F:] + r * gh[:, 2 * F:])
    return (1.0 - z) * n + z * h


def _scan_body(cap_ref, mi_ref, ps_ref, wih_ref, whh_ref, bih_ref, bhh_ref,
               out_ref, h_ref):
    t = pl.program_id(0)

    @pl.when(t == 0)
    def _():
        h_ref[...] = ps_ref[...]

    x = mi_ref[0]
    h = h_ref[...]
    h_new = _gru_math(x, h, wih_ref[...], whh_ref[...], bih_ref[...],
                      bhh_ref[...])
    keep = t < cap_ref[0]
    h2 = jnp.where(keep, h_new, h)
    h_ref[...] = h2
    out_ref[0] = h2


def _msg_scan(mi, ps, wih_t, whh_t, bih, bhh, cap):
    return pl.pallas_call(
        _scan_body,
        grid_spec=pltpu.PrefetchScalarGridSpec(
            num_scalar_prefetch=1,
            grid=(ML,),
            in_specs=[
                pl.BlockSpec((1, NA, F), lambda t, *_: (t, 0, 0)),
                pl.BlockSpec((NA, F), lambda t, *_: (0, 0)),
                pl.BlockSpec((F, 3 * F), lambda t, *_: (0, 0)),
                pl.BlockSpec((F, 3 * F), lambda t, *_: (0, 0)),
                pl.BlockSpec((1, 3 * F), lambda t, *_: (0, 0)),
                pl.BlockSpec((1, 3 * F), lambda t, *_: (0, 0)),
            ],
            out_specs=pl.BlockSpec((1, NA, F), lambda t, *_: (t, 0, 0)),
            scratch_shapes=[pltpu.VMEM((NA, F), jnp.float32)],
        ),
        out_shape=jax.ShapeDtypeStruct((ML, NA, F), jnp.float32),
    )(cap, mi, ps, wih_t, whh_t, bih, bhh)


def _update_body(nl_ref, x_ref, h_ref, wih_ref, whh_ref, bih_ref, bhh_ref,
                 out_ref):
    i = pl.program_id(0)
    rows = h_ref.shape[0]
    x = x_ref[0] + x_ref[1]
    h = h_ref[...]
    h_new = _gru_math(x, h, wih_ref[...], whh_ref[...], bih_ref[...],
                      bhh_ref[...])
    row = jax.lax.broadcasted_iota(jnp.int32, (rows, 1), 0) + i * rows
    out_ref[...] = jnp.where(row < nl_ref[0], h_new, 0.0)


def _link_update(parts, ls_ext, wih_t, whh_t, bih, bhh, nl):
    blk = NLp // 8
    return pl.pallas_call(
        _update_body,
        grid_spec=pltpu.PrefetchScalarGridSpec(
            num_scalar_prefetch=1,
            grid=(8,),
            in_specs=[
                pl.BlockSpec((NC, blk, F), lambda i, *_: (0, i, 0)),
                pl.BlockSpec((blk, F), lambda i, *_: (i, 0)),
                pl.BlockSpec((F, 3 * F), lambda i, *_: (0, 0)),
                pl.BlockSpec((F, 3 * F), lambda i, *_: (0, 0)),
                pl.BlockSpec((1, 3 * F), lambda i, *_: (0, 0)),
                pl.BlockSpec((1, 3 * F), lambda i, *_: (0, 0)),
            ],
            out_specs=pl.BlockSpec((blk, F), lambda i, *_: (i, 0)),
            scratch_shapes=[],
        ),
        out_shape=jax.ShapeDtypeStruct((NLp, F), jnp.float32),
    )(nl, parts, ls_ext, wih_t, whh_t, bih, bhh)


def _scan_ro_body(cap_ref, mi_ref, ps_ref, wih_ref, whh_ref, bih_ref,
                  bhh_ref, w1_ref, b1_ref, w2_ref, b2_ref, wo_ref, bo_ref,
                  out_ref, h_ref):
    t = pl.program_id(0)

    @pl.when(t == 0)
    def _():
        h_ref[...] = ps_ref[...]

    x = mi_ref[0]
    h = h_ref[...]
    h_new = _gru_math(x, h, wih_ref[...], whh_ref[...], bih_ref[...],
                      bhh_ref[...])
    keep = t < cap_ref[0]
    h2 = jnp.where(keep, h_new, h)
    h_ref[...] = h2

    @pl.when(t == ML - 1)
    def _():
        hh = _selu(jax.lax.dot_general(h2, w1_ref[...],
                                       (((1,), (0,)), ((), ())),
                                       preferred_element_type=jnp.float32)
                   + b1_ref[...])
        hh = _selu(jax.lax.dot_general(hh, w2_ref[...],
                                       (((1,), (0,)), ((), ())),
                                       preferred_element_type=jnp.float32)
                   + b2_ref[...])
        out_ref[...] = jax.lax.dot_general(hh, wo_ref[...],
                                           (((1,), (0,)), ((), ())),
                                           preferred_element_type=jnp.float32
                                           ) + bo_ref[...]


def _msg_scan_readout(mi, ps, wih_t, whh_t, bih, bhh, cap,
                      w1_t, b1, w2_t, b2, wo_t, bo):
    return pl.pallas_call(
        _scan_ro_body,
        grid_spec=pltpu.PrefetchScalarGridSpec(
            num_scalar_prefetch=1,
            grid=(ML,),
            in_specs=[
                pl.BlockSpec((1, NA, F), lambda t, *_: (t, 0, 0)),
                pl.BlockSpec((NA, F), lambda t, *_: (0, 0)),
                pl.BlockSpec((F, 3 * F), lambda t, *_: (0, 0)),
                pl.BlockSpec((F, 3 * F), lambda t, *_: (0, 0)),
                pl.BlockSpec((1, 3 * F), lambda t, *_: (0, 0)),
                pl.BlockSpec((1, 3 * F), lambda t, *_: (0, 0)),
                pl.BlockSpec((F, 2 * F), lambda t, *_: (0, 0)),
                pl.BlockSpec((1, 2 * F), lambda t, *_: (0, 0)),
                pl.BlockSpec((2 * F, 2 * F), lambda t, *_: (0, 0)),
                pl.BlockSpec((1, 2 * F), lambda t, *_: (0, 0)),
                pl.BlockSpec((2 * F, F), lambda t, *_: (0, 0)),
                pl.BlockSpec((1, F), lambda t, *_: (0, 0)),
            ],
            out_specs=pl.BlockSpec((NA, F), lambda t, *_: (0, 0)),
            scratch_shapes=[pltpu.VMEM((NA, F), jnp.float32)],
        ),
        out_shape=jax.ShapeDtypeStruct((NA, F), jnp.float32),
    )(cap, mi, ps, wih_t, whh_t, bih, bhh, w1_t, b1, w2_t, b2, wo_t, bo)


def _selu(x):
    alpha = 1.6732632423543772848170429916717
    scale = 1.0507009873554804934193349852946
    return scale * jnp.where(x > 0, x, alpha * (jnp.exp(x) - 1.0))


def _readout_body(ps_ref, w1_ref, b1_ref, w2_ref, b2_ref, wo_ref, bo_ref,
                  out_ref):
    h = _selu(jax.lax.dot_general(ps_ref[...], w1_ref[...],
                                  (((1,), (0,)), ((), ())),
                                  preferred_element_type=jnp.float32)
              + b1_ref[...])
    h = _selu(jax.lax.dot_general(h, w2_ref[...], (((1,), (0,)), ((), ())),
                                  preferred_element_type=jnp.float32)
              + b2_ref[...])
    out_ref[...] = jax.lax.dot_general(h, wo_ref[...],
                                       (((1,), (0,)), ((), ())),
                                       preferred_element_type=jnp.float32) \
        + bo_ref[...]


def _readout(ps, w1_t, b1, w2_t, b2, wo_t, bo):
    return pl.pallas_call(
        _readout_body,
        out_shape=jax.ShapeDtypeStruct((NA, F), jnp.float32),
    )(ps, w1_t, b1, w2_t, b2, wo_t, bo)


def _pad_body(ls_ref, out_ref):
    nl = ls_ref.shape[0]
    out_ref[:nl, :] = ls_ref[...]
    out_ref[nl:, :] = jnp.zeros_like(out_ref[nl:, :])


def _pad_links(ls):
    return pl.pallas_call(
        _pad_body,
        out_shape=jax.ShapeDtypeStruct((NLp, F), jnp.float32),
    )(ls)


def kernel(link_state, path_state, link_id, path_id, sequence, num_actions,
           W_ih_m, W_hh_m, b_ih_m, b_hh_m, W_ih_u, W_hh_u, b_ih_u, b_hh_u,
           W_r1, b_r1, W_r2, b_r2, W_out, b_out):
    T = 4

    lid = link_id.astype(jnp.int32)
    pid = path_id.astype(jnp.int32)
    seq = sequence.astype(jnp.int32)

    wm_ih = W_ih_m.T
    wm_hh = W_hh_m.T
    wu_ih = W_ih_u.T
    wu_hh = W_hh_u.T
    bm_ih = b_ih_m.reshape(1, -1)
    bm_hh = b_hh_m.reshape(1, -1)
    bu_ih = b_ih_u.reshape(1, -1)
    bu_hh = b_hh_u.reshape(1, -1)
    w1_t = W_r1.T
    w2_t = W_r2.T
    wo_t = jnp.pad(W_out.T, ((0, 0), (0, F - W_out.shape[0])))
    b1 = b_r1.reshape(1, -1)
    b2 = b_r2.reshape(1, -1)
    bo = jnp.pad(b_out.reshape(1, -1), ((0, 0), (0, F - b_out.shape[0])))

    src, flat, capm = _sc_invert(pid, seq, lid)
    src2 = src.reshape(S // 128, 128)
    flat2 = flat.reshape(E // 128, 128)
    lid2 = lid.reshape(E // 128, 128)
    cap = jnp.minimum(jnp.max(capm) + 1, num_actions).astype(jnp.int32)
    cap_arr = cap.reshape(1)
    nl_arr = jnp.full((1,), NL, jnp.int32)
    zeros_hbm = jnp.zeros((NLp, F), jnp.float32)

    ls_ext = _pad_links(link_state)
    ps = path_state
    for r in range(T - 1):
        mi = _sc_gather(ls_ext, src2).reshape(ML, NA, F)
        m_seq = _msg_scan(mi, ps, wm_ih, wm_hh, bm_ih, bm_hh, cap_arr)
        ps = m_seq[ML - 1]
        parts = _sc_agg(m_seq.reshape(S, F), flat2, lid2, zeros_hbm)
        ls_ext = _link_update(parts, ls_ext, wu_ih, wu_hh, bu_ih, bu_hh,
                              nl_arr)
    mi = _sc_gather(ls_ext, src2).reshape(ML, NA, F)
    out = _msg_scan_readout(mi, ps, wm_ih, wm_hh, bm_ih, bm_hh, cap_arr,
                            w1_t, b1, w2_t, b2, wo_t, bo)
    return out[:, :1]


# speedup vs baseline: 1.0484x; 1.0032x over previous
import functools

import jax
import jax.numpy as jnp
from jax import lax
from jax.experimental import pallas as pl
from jax.experimental.pallas import tpu as pltpu
from jax.experimental.pallas import tpu_sc as plsc

ML = 8
F = 128
PAD = 112

NL = 10000
NA = 4096
E = 32768
S = ML * NA
NLp = NL + PAD
NC = 2
NS = 16
NW = NC * NS
EPW = E // NW
SPW = S // NW
RPS = NLp // NS

_SC_MESH = plsc.VectorSubcoreMesh(core_axis_name="c", subcore_axis_name="s")
_SC_PARAMS = pltpu.CompilerParams(needs_layout_passes=False)


def _ds(base, size):
    return pl.ds(pl.multiple_of(base, 8), size)


def _inv_body(pid_hbm, seq_hbm, lid_hbm, src_hbm, flat_hbm, cap_hbm,
              pid_v, seq_v, lid_v, src_v, flat_v, cap_v):
    w = lax.axis_index("s") * NC + lax.axis_index("c")
    lo = w * SPW
    pltpu.sync_copy(pid_hbm, pid_v)
    pltpu.sync_copy(seq_hbm, seq_v)
    pltpu.sync_copy(lid_hbm, lid_v)

    def init_body(j, carry):
        spread = (lax.iota(jnp.int32, 16) + j * 16) & 63
        src_v[pl.ds(j * 16, 16)] = NL + spread
        return carry

    lax.fori_loop(0, SPW // 16, init_body, 0)

    def scan_body(i, carry):
        sq = seq_v[pl.ds(i * 16, 16)]
        f = sq * NA + pid_v[pl.ds(i * 16, 16)]
        l = lid_v[pl.ds(i * 16, 16)]
        fl = f - lo
        m = (fl >= 0) & (fl < SPW)
        plsc.store_scatter(src_v, [fl], l, mask=m)
        return carry

    lax.fori_loop(0, E // 16, scan_body, 0)
    pltpu.sync_copy(src_v, src_hbm.at[_ds(lo, SPW)])

    ebase = w * EPW

    def flat_body(k, mx):
        sq = seq_v[pl.ds(ebase + k * 16, 16)]
        f = sq * NA + pid_v[pl.ds(ebase + k * 16, 16)]
        flat_v[pl.ds(k * 16, 16)] = f
        return jnp.maximum(mx, sq)

    mx = lax.fori_loop(0, EPW // 16, flat_body, jnp.zeros((16,), jnp.int32))
    pltpu.sync_copy(flat_v, flat_hbm.at[_ds(ebase, EPW)])
    for r in range(8):
        for j in range(8):
            cap_v[r, pl.ds(j * 16, 16)] = mx
    pltpu.sync_copy(cap_v, cap_hbm.at[_ds(w * 8, 8)])


_sc_invert = functools.partial(
    pl.kernel, _inv_body, mesh=_SC_MESH, compiler_params=_SC_PARAMS,
    out_type=(jax.ShapeDtypeStruct((S,), jnp.int32),
              jax.ShapeDtypeStruct((E,), jnp.int32),
              jax.ShapeDtypeStruct((NW * 8, 128), jnp.int32)),
    scratch_types=[pltpu.VMEM((E,), jnp.int32),
                   pltpu.VMEM((E,), jnp.int32),
                   pltpu.VMEM((E,), jnp.int32),
                   pltpu.VMEM((SPW,), jnp.int32),
                   pltpu.VMEM((EPW,), jnp.int32),
                   pltpu.VMEM((8, 128), jnp.int32)],
)()


def _gat_body(ls_hbm, src2_hbm, mi_hbm, idx_v, rows_a, rows_b,
              sg_a, sg_b, sw_a, sw_b):
    w = lax.axis_index("s") * NC + lax.axis_index("c")
    pltpu.sync_copy(src2_hbm.at[_ds(w * 8, 8)], idx_v)
    rows = (rows_a, rows_b)
    sg = (sg_a, sg_b)
    sw = (sw_a, sw_b)
    gd = [None, None]
    wd = [None, None]
    gd[0] = pltpu.async_copy(ls_hbm.at[idx_v.at[0]], rows_a, sg_a)
    for cc in range(8):
        b = cc & 1
        gd[b].wait()
        if cc + 1 < 8:
            if cc >= 1:
                wd[1 - b].wait()
            gd[1 - b] = pltpu.async_copy(ls_hbm.at[idx_v.at[cc + 1]],
                                         rows[1 - b], sg[1 - b])
        wd[b] = pltpu.async_copy(rows[b],
                                 mi_hbm.at[_ds(w * SPW + cc * 128, 128)],
                                 sw[b])
    wd[0].wait()
    wd[1].wait()


_sc_gather = functools.partial(
    pl.kernel, _gat_body, mesh=_SC_MESH, compiler_params=_SC_PARAMS,
    out_type=jax.ShapeDtypeStruct((S, F), jnp.float32),
    scratch_types=[pltpu.VMEM((8, 128), jnp.int32),
                   pltpu.VMEM((128, F), jnp.float32),
                   pltpu.VMEM((128, F), jnp.float32),
                   pltpu.SemaphoreType.DMA, pltpu.SemaphoreType.DMA,
                   pltpu.SemaphoreType.DMA, pltpu.SemaphoreType.DMA],
)()


def _agg_body(mseq_hbm, flat2_hbm, lid2_hbm, zeros_hbm, parts_hbm,
              fidx, lidx, rows_a, rows_b, sp, sg_a, sg_b, sw_a, sw_b):
    c = lax.axis_index("c")
    s = lax.axis_index("s")
    w = s * NC + c
    pltpu.sync_copy(zeros_hbm.at[_ds(s * RPS, RPS)],
                    sp.at[_ds(s * RPS, RPS)])
    pltpu.sync_copy(flat2_hbm.at[_ds(w * 8, 8)], fidx)
    pltpu.sync_copy(lid2_hbm.at[_ds(w * 8, 8)], lidx)
    plsc.subcore_barrier()
    rows = (rows_a, rows_b)
    sg = (sg_a, sg_b)
    sw = (sw_a, sw_b)
    gd = [None, None]
    wd = [None, None]
    gd[0] = pltpu.async_copy(mseq_hbm.at[fidx.at[0]], rows_a, sg_a)
    for cc in range(8):
        b = cc & 1
        gd[b].wait()
        if cc + 1 < 8:
            if cc >= 1:
                wd[1 - b].wait()
            gd[1 - b] = pltpu.async_copy(mseq_hbm.at[fidx.at[cc + 1]],
                                         rows[1 - b], sg[1 - b])
        wd[b] = pltpu.async_copy(rows[b], sp.at[lidx.at[cc]], sw[b],
                                 add=True)
    wd[0].wait()
    wd[1].wait()
    plsc.subcore_barrier()
    pltpu.sync_copy(sp.at[_ds(s * RPS, RPS)],
                    parts_hbm.at[c, _ds(s * RPS, RPS)])


_sc_agg = functools.partial(
    pl.kernel, _agg_body, mesh=_SC_MESH, compiler_params=_SC_PARAMS,
    out_type=jax.ShapeDtypeStruct((NC, NLp, F), jnp.float32),
    scratch_types=[pltpu.VMEM((8, 128), jnp.int32),
                   pltpu.VMEM((8, 128), jnp.int32),
                   pltpu.VMEM((128, F), jnp.float32),
                   pltpu.VMEM((128, F), jnp.float32),
                   pltpu.VMEM_SHARED((NLp, F), jnp.float32),
                   pltpu.SemaphoreType.DMA, pltpu.SemaphoreType.DMA,
                   pltpu.SemaphoreType.DMA, pltpu.SemaphoreType.DMA],
)()


def _gru_math(x, h, gi_w, gh_w, b_ih, b_hh):
    gi = jax.lax.dot_general(x, gi_w, (((1,), (0,)), ((), ())),
                             preferred_element_type=jnp.float32) + b_ih
    gh = jax.lax.dot_general(h, gh_w, (((1,), (0,)), ((), ())),
                             preferred_element_type=jnp.float32) + b_hh
    r = jax.nn.sigmoid(gi[:, :F] + gh[:, :F])
    z = jax.nn.sigmoid(gi[:, F:2 * F] + gh[:, F:2 * F])
    n = jnp.tanh(gi[:, 2 * F:] + r * gh[:, 2 * F:])
    return (1.0 - z) * n + z * h


def _scan_body(cap_ref, mi_ref, ps_ref, wih_ref, whh_ref, bih_ref, bhh_ref,
               out_ref, psout_ref, h_ref):
    t = pl.program_id(0)

    @pl.when(t == 0)
    def _():
        h_ref[...] = ps_ref[...]

    x = mi_ref[0]
    h = h_ref[...]
    h_new = _gru_math(x, h, wih_ref[...], whh_ref[...], bih_ref[...],
                      bhh_ref[...])
    keep = t < cap_ref[0]
    h2 = jnp.where(keep, h_new, h)
    h_ref[...] = h2
    out_ref[0] = h2

    @pl.when(t == ML - 1)
    def _():
        psout_ref[...] = h2


def _msg_scan(mi, ps, wih_t, whh_t, bih, bhh, cap):
    return pl.pallas_call(
        _scan_body,
        grid_spec=pltpu.PrefetchScalarGridSpec(
            num_scalar_prefetch=1,
            grid=(ML,),
            in_specs=[
                pl.BlockSpec((1, NA, F), lambda t, *_: (t, 0, 0)),
                pl.BlockSpec((NA, F), lambda t, *_: (0, 0)),
                pl.BlockSpec((F, 3 * F), lambda t, *_: (0, 0)),
                pl.BlockSpec((F, 3 * F), lambda t, *_: (0, 0)),
                pl.BlockSpec((1, 3 * F), lambda t, *_: (0, 0)),
                pl.BlockSpec((1, 3 * F), lambda t, *_: (0, 0)),
            ],
            out_specs=[pl.BlockSpec((1, NA, F), lambda t, *_: (t, 0, 0)),
                       pl.BlockSpec((NA, F), lambda t, *_: (0, 0))],
            scratch_shapes=[pltpu.VMEM((NA, F), jnp.float32)],
        ),
        out_shape=[jax.ShapeDtypeStruct((ML, NA, F), jnp.float32),
                   jax.ShapeDtypeStruct((NA, F), jnp.float32)],
    )(cap, mi, ps, wih_t, whh_t, bih, bhh)


def _update_body(nl_ref, x_ref, h_ref, wih_ref, whh_ref, bih_ref, bhh_ref,
                 out_ref):
    i = pl.program_id(0)
    rows = h_ref.shape[0]
    x = x_ref[0] + x_ref[1]
    h = h_ref[...]
    h_new = _gru_math(x, h, wih_ref[...], whh_ref[...], bih_ref[...],
                      bhh_ref[...])
    row = jax.lax.broadcasted_iota(jnp.int32, (rows, 1), 0) + i * rows
    out_ref[...] = jnp.where(row < nl_ref[0], h_new, 0.0)


def _link_update(parts, ls_ext, wih_t, whh_t, bih, bhh, nl):
    blk = NLp // 8
    return pl.pallas_call(
        _update_body,
        grid_spec=pltpu.PrefetchScalarGridSpec(
            num_scalar_prefetch=1,
            grid=(8,),
            in_specs=[
                pl.BlockSpec((NC, blk, F), lambda i, *_: (0, i, 0)),
                pl.BlockSpec((blk, F), lambda i, *_: (i, 0)),
                pl.BlockSpec((F, 3 * F), lambda i, *_: (0, 0)),
                pl.BlockSpec((F, 3 * F), lambda i, *_: (0, 0)),
                pl.BlockSpec((1, 3 * F), lambda i, *_: (0, 0)),
                pl.BlockSpec((1, 3 * F), lambda i, *_: (0, 0)),
            ],
            out_specs=pl.BlockSpec((blk, F), lambda i, *_: (i, 0)),
            scratch_shapes=[],
        ),
        out_shape=jax.ShapeDtypeStruct((NLp, F), jnp.float32),
    )(nl, parts, ls_ext, wih_t, whh_t, bih, bhh)


def _scan_ro_body(cap_ref, mi_ref, ps_ref, wih_ref, whh_ref, bih_ref,
                  bhh_ref, w1_ref, b1_ref, w2_ref, b2_ref, wo_ref, bo_ref,
                  out_ref, h_ref):
    t = pl.program_id(0)

    @pl.when(t == 0)
    def _():
        h_ref[...] = ps_ref[...]

    x = mi_ref[0]
    h = h_ref[...]
    h_new = _gru_math(x, h, wih_ref[...], whh_ref[...], bih_ref[...],
                      bhh_ref[...])
    keep = t < cap_ref[0]
    h2 = jnp.where(keep, h_new, h)
    h_ref[...] = h2

    @pl.when(t == ML - 1)
    def _():
        hh = _selu(jax.lax.dot_general(h2, w1_ref[...],
                                       (((1,), (0,)), ((), ())),
                                       preferred_element_type=jnp.float32)
                   + b1_ref[...])
        hh = _selu(jax.lax.dot_general(hh, w2_ref[...],
                                       (((1,), (0,)), ((), ())),
                                       preferred_element_type=jnp.float32)
                   + b2_ref[...])
        res = jax.lax.dot_general(hh, wo_ref[...],
                                  (((1,), (0,)), ((), ())),
                                  preferred_element_type=jnp.float32
                                  ) + bo_ref[...]
        out_ref[...] = res[:, :1]


def _msg_scan_readout(mi, ps, wih_t, whh_t, bih, bhh, cap,
                      w1_t, b1, w2_t, b2, wo_t, bo):
    return pl.pallas_call(
        _scan_ro_body,
        grid_spec=pltpu.PrefetchScalarGridSpec(
            num_scalar_prefetch=1,
            grid=(ML,),
            in_specs=[
                pl.BlockSpec((1, NA, F), lambda t, *_: (t, 0, 0)),
                pl.BlockSpec((NA, F), lambda t, *_: (0, 0)),
                pl.BlockSpec((F, 3 * F), lambda t, *_: (0, 0)),
                pl.BlockSpec((F, 3 * F), lambda t, *_: (0, 0)),
                pl.BlockSpec((1, 3 * F), lambda t, *_: (0, 0)),
                pl.BlockSpec((1, 3 * F), lambda t, *_: (0, 0)),
                pl.BlockSpec((F, 2 * F), lambda t, *_: (0, 0)),
                pl.BlockSpec((1, 2 * F), lambda t, *_: (0, 0)),
                pl.BlockSpec((2 * F, 2 * F), lambda t, *_: (0, 0)),
                pl.BlockSpec((1, 2 * F), lambda t, *_: (0, 0)),
                pl.BlockSpec((2 * F, F), lambda t, *_: (0, 0)),
                pl.BlockSpec((1, F), lambda t, *_: (0, 0)),
            ],
            out_specs=pl.BlockSpec((NA, 1), lambda t, *_: (0, 0)),
            scratch_shapes=[pltpu.VMEM((NA, F), jnp.float32)],
        ),
        out_shape=jax.ShapeDtypeStruct((NA, 1), jnp.float32),
    )(cap, mi, ps, wih_t, whh_t, bih, bhh, w1_t, b1, w2_t, b2, wo_t, bo)


def _selu(x):
    alpha = 1.6732632423543772848170429916717
    scale = 1.0507009873554804934193349852946
    return scale * jnp.where(x > 0, x, alpha * (jnp.exp(x) - 1.0))


def _readout_body(ps_ref, w1_ref, b1_ref, w2_ref, b2_ref, wo_ref, bo_ref,
                  out_ref):
    h = _selu(jax.lax.dot_general(ps_ref[...], w1_ref[...],
                                  (((1,), (0,)), ((), ())),
                                  preferred_element_type=jnp.float32)
              + b1_ref[...])
    h = _selu(jax.lax.dot_general(h, w2_ref[...], (((1,), (0,)), ((), ())),
                                  preferred_element_type=jnp.float32)
              + b2_ref[...])
    out_ref[...] = jax.lax.dot_general(h, wo_ref[...],
                                       (((1,), (0,)), ((), ())),
                                       preferred_element_type=jnp.float32) \
        + bo_ref[...]


def _readout(ps, w1_t, b1, w2_t, b2, wo_t, bo):
    return pl.pallas_call(
        _readout_body,
        out_shape=jax.ShapeDtypeStruct((NA, F), jnp.float32),
    )(ps, w1_t, b1, w2_t, b2, wo_t, bo)


def _pad_body(ls_ref, out_ref):
    nl = ls_ref.shape[0]
    out_ref[:nl, :] = ls_ref[...]
    out_ref[nl:, :] = jnp.zeros_like(out_ref[nl:, :])


def _pad_links(ls):
    return pl.pallas_call(
        _pad_body,
        out_shape=jax.ShapeDtypeStruct((NLp, F), jnp.float32),
    )(ls)


def kernel(link_state, path_state, link_id, path_id, sequence, num_actions,
           W_ih_m, W_hh_m, b_ih_m, b_hh_m, W_ih_u, W_hh_u, b_ih_u, b_hh_u,
           W_r1, b_r1, W_r2, b_r2, W_out, b_out):
    T = 4

    lid = link_id.astype(jnp.int32)
    pid = path_id.astype(jnp.int32)
    seq = sequence.astype(jnp.int32)

    wm_ih = W_ih_m.T
    wm_hh = W_hh_m.T
    wu_ih = W_ih_u.T
    wu_hh = W_hh_u.T
    bm_ih = b_ih_m.reshape(1, -1)
    bm_hh = b_hh_m.reshape(1, -1)
    bu_ih = b_ih_u.reshape(1, -1)
    bu_hh = b_hh_u.reshape(1, -1)
    w1_t = W_r1.T
    w2_t = W_r2.T
    wo_t = jnp.pad(W_out.T, ((0, 0), (0, F - W_out.shape[0])))
    b1 = b_r1.reshape(1, -1)
    b2 = b_r2.reshape(1, -1)
    bo = jnp.pad(b_out.reshape(1, -1), ((0, 0), (0, F - b_out.shape[0])))

    src, flat, capm = _sc_invert(pid, seq, lid)
    src2 = src.reshape(S // 128, 128)
    flat2 = flat.reshape(E // 128, 128)
    lid2 = lid.reshape(E // 128, 128)
    cap = jnp.minimum(jnp.max(capm) + 1, num_actions).astype(jnp.int32)
    cap_arr = cap.reshape(1)
    nl_arr = jnp.full((1,), NL, jnp.int32)
    zeros_hbm = jnp.zeros((NLp, F), jnp.float32)

    ls_ext = _pad_links(link_state)
    ps = path_state
    for r in range(T - 1):
        mi = _sc_gather(ls_ext, src2).reshape(ML, NA, F)
        m_seq, ps = _msg_scan(mi, ps, wm_ih, wm_hh, bm_ih, bm_hh, cap_arr)
        parts = _sc_agg(m_seq.reshape(S, F), flat2, lid2, zeros_hbm)
        ls_ext = _link_update(parts, ls_ext, wu_ih, wu_hh, bu_ih, bu_hh,
                              nl_arr)
    mi = _sc_gather(ls_ext, src2).reshape(ML, NA, F)
    return _msg_scan_readout(mi, ps, wm_ih, wm_hh, bm_ih, bm_hh, cap_arr,
                             w1_t, b1, w2_t, b2, wo_t, bo)


# speedup vs baseline: 1.0924x; 1.0420x over previous
import functools

import jax
import jax.numpy as jnp
from jax import lax
from jax.experimental import pallas as pl
from jax.experimental.pallas import tpu as pltpu
from jax.experimental.pallas import tpu_sc as plsc

ML = 8
F = 128
PAD = 112

NL = 10000
NA = 4096
E = 32768
S = ML * NA
NLp = NL + PAD
NC = 2
NS = 16
NW = NC * NS
EPW = E // NW
SPW = S // NW
RPS = NLp // NS

_SC_MESH = plsc.VectorSubcoreMesh(core_axis_name="c", subcore_axis_name="s")
_SC_PARAMS = pltpu.CompilerParams(needs_layout_passes=False)


def _ds(base, size):
    return pl.ds(pl.multiple_of(base, 8), size)


def _inv_body(pid_hbm, seq_hbm, lid_hbm, ls_hbm, src_hbm, flat_hbm, cap_hbm,
              mi_hbm, pid_v, seq_v, lid_v, src_v, flat_v, cap_v,
              rows_a, rows_b, sg_a, sg_b, sw_a, sw_b):
    w = lax.axis_index("s") * NC + lax.axis_index("c")
    lo = w * SPW
    pltpu.sync_copy(pid_hbm, pid_v)
    pltpu.sync_copy(seq_hbm, seq_v)
    pltpu.sync_copy(lid_hbm, lid_v)

    def init_body(j, carry):
        spread = (lax.iota(jnp.int32, 16) + j * 16) & 63
        src_v[pl.ds(j * 16, 16)] = NL + spread
        return carry

    lax.fori_loop(0, SPW // 16, init_body, 0)

    def scan_body(i, carry):
        for u in range(2):
            base = i * 32 + u * 16
            sq = seq_v[pl.ds(base, 16)]
            f = sq * NA + pid_v[pl.ds(base, 16)]
            l = lid_v[pl.ds(base, 16)]
            fl = f - lo
            m = (fl >= 0) & (fl < SPW)
            plsc.store_scatter(src_v, [fl], l, mask=m)
        return carry

    lax.fori_loop(0, E // 32, scan_body, 0)
    pltpu.sync_copy(src_v, src_hbm.at[_ds(lo, SPW)])

    ebase = w * EPW

    def flat_body(k, mx):
        sq = seq_v[pl.ds(ebase + k * 16, 16)]
        f = sq * NA + pid_v[pl.ds(ebase + k * 16, 16)]
        flat_v[pl.ds(k * 16, 16)] = f
        return jnp.maximum(mx, sq)

    mx = lax.fori_loop(0, EPW // 16, flat_body, jnp.zeros((16,), jnp.int32))
    pltpu.sync_copy(flat_v, flat_hbm.at[_ds(ebase, EPW)])
    for r in range(8):
        for j in range(8):
            cap_v[r, pl.ds(j * 16, 16)] = mx
    pltpu.sync_copy(cap_v, cap_hbm.at[_ds(w * 8, 8)])
    rows = (rows_a, rows_b)
    sg = (sg_a, sg_b)
    sw = (sw_a, sw_b)
    gd = [None, None]
    wd = [None, None]
    gd[0] = pltpu.async_copy(ls_hbm.at[src_v.at[pl.ds(0, 64)]], rows_a,
                             sg_a)
    for cc in range(16):
        b = cc & 1
        gd[b].wait()
        if cc + 1 < 16:
            if cc >= 1:
                wd[1 - b].wait()
            gd[1 - b] = pltpu.async_copy(
                ls_hbm.at[src_v.at[pl.ds((cc + 1) * 64, 64)]],
                rows[1 - b], sg[1 - b])
        wd[b] = pltpu.async_copy(rows[b],
                                 mi_hbm.at[_ds(w * SPW + cc * 64, 64)],
                                 sw[b])
    wd[0].wait()
    wd[1].wait()


_sc_invert = functools.partial(
    pl.kernel, _inv_body, mesh=_SC_MESH, compiler_params=_SC_PARAMS,
    out_type=(jax.ShapeDtypeStruct((S,), jnp.int32),
              jax.ShapeDtypeStruct((E,), jnp.int32),
              jax.ShapeDtypeStruct((NW * 8, 128), jnp.int32),
              jax.ShapeDtypeStruct((S, F), jnp.float32)),
    scratch_types=[pltpu.VMEM((E,), jnp.int32),
                   pltpu.VMEM((E,), jnp.int32),
                   pltpu.VMEM((E,), jnp.int32),
                   pltpu.VMEM((SPW,), jnp.int32),
                   pltpu.VMEM((EPW,), jnp.int32),
                   pltpu.VMEM((8, 128), jnp.int32),
                   pltpu.VMEM((64, F), jnp.float32),
                   pltpu.VMEM((64, F), jnp.float32),
                   pltpu.SemaphoreType.DMA, pltpu.SemaphoreType.DMA,
                   pltpu.SemaphoreType.DMA, pltpu.SemaphoreType.DMA],
)()


def _gat_body(ls_hbm, src2_hbm, mi_hbm, idx_v, rows_a, rows_b,
              sg_a, sg_b, sw_a, sw_b):
    w = lax.axis_index("s") * NC + lax.axis_index("c")
    pltpu.sync_copy(src2_hbm.at[_ds(w * 8, 8)], idx_v)
    rows = (rows_a, rows_b)
    sg = (sg_a, sg_b)
    sw = (sw_a, sw_b)
    gd = [None, None]
    wd = [None, None]
    gd[0] = pltpu.async_copy(ls_hbm.at[idx_v.at[0]], rows_a, sg_a)
    for cc in range(8):
        b = cc & 1
        gd[b].wait()
        if cc + 1 < 8:
            if cc >= 1:
                wd[1 - b].wait()
            gd[1 - b] = pltpu.async_copy(ls_hbm.at[idx_v.at[cc + 1]],
                                         rows[1 - b], sg[1 - b])
        wd[b] = pltpu.async_copy(rows[b],
                                 mi_hbm.at[_ds(w * SPW + cc * 128, 128)],
                                 sw[b])
    wd[0].wait()
    wd[1].wait()


_sc_gather = functools.partial(
    pl.kernel, _gat_body, mesh=_SC_MESH, compiler_params=_SC_PARAMS,
    out_type=jax.ShapeDtypeStruct((S, F), jnp.float32),
    scratch_types=[pltpu.VMEM((8, 128), jnp.int32),
                   pltpu.VMEM((128, F), jnp.float32),
                   pltpu.VMEM((128, F), jnp.float32),
                   pltpu.SemaphoreType.DMA, pltpu.SemaphoreType.DMA,
                   pltpu.SemaphoreType.DMA, pltpu.SemaphoreType.DMA],
)()


def _agg_body(mseq_hbm, flat2_hbm, lid2_hbm, zeros_hbm, parts_hbm,
              fidx, lidx, rows_a, rows_b, sp, sg_a, sg_b, sw_a, sw_b):
    c = lax.axis_index("c")
    s = lax.axis_index("s")
    w = s * NC + c
    pltpu.sync_copy(zeros_hbm.at[_ds(s * RPS, RPS)],
                    sp.at[_ds(s * RPS, RPS)])
    pltpu.sync_copy(flat2_hbm.at[_ds(w * 8, 8)], fidx)
    pltpu.sync_copy(lid2_hbm.at[_ds(w * 8, 8)], lidx)
    plsc.subcore_barrier()
    rows = (rows_a, rows_b)
    sg = (sg_a, sg_b)
    sw = (sw_a, sw_b)
    gd = [None, None]
    wd = [None, None]
    gd[0] = pltpu.async_copy(mseq_hbm.at[fidx.at[0]], rows_a, sg_a)
    for cc in range(8):
        b = cc & 1
        gd[b].wait()
        if cc + 1 < 8:
            if cc >= 1:
                wd[1 - b].wait()
            gd[1 - b] = pltpu.async_copy(mseq_hbm.at[fidx.at[cc + 1]],
                                         rows[1 - b], sg[1 - b])
        wd[b] = pltpu.async_copy(rows[b], sp.at[lidx.at[cc]], sw[b],
                                 add=True)
    wd[0].wait()
    wd[1].wait()
    plsc.subcore_barrier()
    pltpu.sync_copy(sp.at[_ds(s * RPS, RPS)],
                    parts_hbm.at[c, _ds(s * RPS, RPS)])


_sc_agg = functools.partial(
    pl.kernel, _agg_body, mesh=_SC_MESH, compiler_params=_SC_PARAMS,
    out_type=jax.ShapeDtypeStruct((NC, NLp, F), jnp.float32),
    scratch_types=[pltpu.VMEM((8, 128), jnp.int32),
                   pltpu.VMEM((8, 128), jnp.int32),
                   pltpu.VMEM((128, F), jnp.float32),
                   pltpu.VMEM((128, F), jnp.float32),
                   pltpu.VMEM_SHARED((NLp, F), jnp.float32),
                   pltpu.SemaphoreType.DMA, pltpu.SemaphoreType.DMA,
                   pltpu.SemaphoreType.DMA, pltpu.SemaphoreType.DMA],
)()


def _gru_math(x, h, gi_w, gh_w, b_ih, b_hh):
    gi = jax.lax.dot_general(x, gi_w, (((1,), (0,)), ((), ())),
                             preferred_element_type=jnp.float32) + b_ih
    gh = jax.lax.dot_general(h, gh_w, (((1,), (0,)), ((), ())),
                             preferred_element_type=jnp.float32) + b_hh
    r = jax.nn.sigmoid(gi[:, :F] + gh[:, :F])
    z = jax.nn.sigmoid(gi[:, F:2 * F] + gh[:, F:2 * F])
    n = jnp.tanh(gi[:, 2 * F:] + r * gh[:, 2 * F:])
    return (1.0 - z) * n + z * h


def _scan_body(cap_ref, mi_ref, ps_ref, wih_ref, whh_ref, bih_ref, bhh_ref,
               out_ref, psout_ref, h_ref):
    t = pl.program_id(0)

    @pl.when(t == 0)
    def _():
        h_ref[...] = ps_ref[...]

    x = mi_ref[0]
    h = h_ref[...]
    h_new = _gru_math(x, h, wih_ref[...], whh_ref[...], bih_ref[...],
                      bhh_ref[...])
    keep = t < cap_ref[0]
    h2 = jnp.where(keep, h_new, h)
    h_ref[...] = h2
    out_ref[0] = h2

    @pl.when(t == ML - 1)
    def _():
        psout_ref[...] = h2


def _msg_scan(mi, ps, wih_t, whh_t, bih, bhh, cap):
    return pl.pallas_call(
        _scan_body,
        grid_spec=pltpu.PrefetchScalarGridSpec(
            num_scalar_prefetch=1,
            grid=(ML,),
            in_specs=[
                pl.BlockSpec((1, NA, F), lambda t, *_: (t, 0, 0)),
                pl.BlockSpec((NA, F), lambda t, *_: (0, 0)),
                pl.BlockSpec((F, 3 * F), lambda t, *_: (0, 0)),
                pl.BlockSpec((F, 3 * F), lambda t, *_: (0, 0)),
                pl.BlockSpec((1, 3 * F), lambda t, *_: (0, 0)),
                pl.BlockSpec((1, 3 * F), lambda t, *_: (0, 0)),
            ],
            out_specs=[pl.BlockSpec((1, NA, F), lambda t, *_: (t, 0, 0)),
                       pl.BlockSpec((NA, F), lambda t, *_: (0, 0))],
            scratch_shapes=[pltpu.VMEM((NA, F), jnp.float32)],
        ),
        out_shape=[jax.ShapeDtypeStruct((ML, NA, F), jnp.float32),
                   jax.ShapeDtypeStruct((NA, F), jnp.float32)],
    )(cap, mi, ps, wih_t, whh_t, bih, bhh)


def _update_body(nl_ref, x_ref, h_ref, wih_ref, whh_ref, bih_ref, bhh_ref,
                 out_ref):
    i = pl.program_id(0)
    rows = h_ref.shape[0]
    x = x_ref[0] + x_ref[1]
    h = h_ref[...]
    h_new = _gru_math(x, h, wih_ref[...], whh_ref[...], bih_ref[...],
                      bhh_ref[...])
    row = jax.lax.broadcasted_iota(jnp.int32, (rows, 1), 0) + i * rows
    out_ref[...] = jnp.where(row < nl_ref[0], h_new, 0.0)


def _link_update(parts, ls_ext, wih_t, whh_t, bih, bhh, nl):
    blk = NLp // 8
    return pl.pallas_call(
        _update_body,
        grid_spec=pltpu.PrefetchScalarGridSpec(
            num_scalar_prefetch=1,
            grid=(8,),
            in_specs=[
                pl.BlockSpec((NC, blk, F), lambda i, *_: (0, i, 0)),
                pl.BlockSpec((blk, F), lambda i, *_: (i, 0)),
                pl.BlockSpec((F, 3 * F), lambda i, *_: (0, 0)),
                pl.BlockSpec((F, 3 * F), lambda i, *_: (0, 0)),
                pl.BlockSpec((1, 3 * F), lambda i, *_: (0, 0)),
                pl.BlockSpec((1, 3 * F), lambda i, *_: (0, 0)),
            ],
            out_specs=pl.BlockSpec((blk, F), lambda i, *_: (i, 0)),
            scratch_shapes=[],
        ),
        out_shape=jax.ShapeDtypeStruct((NLp, F), jnp.float32),
    )(nl, parts, ls_ext, wih_t, whh_t, bih, bhh)


def _scan_ro_body(cap_ref, mi_ref, ps_ref, wih_ref, whh_ref, bih_ref,
                  bhh_ref, w1_ref, b1_ref, w2_ref, b2_ref, wo_ref, bo_ref,
                  out_ref, h_ref):
    t = pl.program_id(0)

    @pl.when(t == 0)
    def _():
        h_ref[...] = ps_ref[...]

    x = mi_ref[0]
    h = h_ref[...]
    h_new = _gru_math(x, h, wih_ref[...], whh_ref[...], bih_ref[...],
                      bhh_ref[...])
    keep = t < cap_ref[0]
    h2 = jnp.where(keep, h_new, h)
    h_ref[...] = h2

    @pl.when(t == ML - 1)
    def _():
        hh = _selu(jax.lax.dot_general(h2, w1_ref[...],
                                       (((1,), (0,)), ((), ())),
                                       preferred_element_type=jnp.float32)
                   + b1_ref[...])
        hh = _selu(jax.lax.dot_general(hh, w2_ref[...],
                                       (((1,), (0,)), ((), ())),
                                       preferred_element_type=jnp.float32)
                   + b2_ref[...])
        res = jax.lax.dot_general(hh, wo_ref[...],
                                  (((1,), (0,)), ((), ())),
                                  preferred_element_type=jnp.float32
                                  ) + bo_ref[...]
        out_ref[...] = res[:, :1]


def _msg_scan_readout(mi, ps, wih_t, whh_t, bih, bhh, cap,
                      w1_t, b1, w2_t, b2, wo_t, bo):
    return pl.pallas_call(
        _scan_ro_body,
        grid_spec=pltpu.PrefetchScalarGridSpec(
            num_scalar_prefetch=1,
            grid=(ML,),
            in_specs=[
                pl.BlockSpec((1, NA, F), lambda t, *_: (t, 0, 0)),
                pl.BlockSpec((NA, F), lambda t, *_: (0, 0)),
                pl.BlockSpec((F, 3 * F), lambda t, *_: (0, 0)),
                pl.BlockSpec((F, 3 * F), lambda t, *_: (0, 0)),
                pl.BlockSpec((1, 3 * F), lambda t, *_: (0, 0)),
                pl.BlockSpec((1, 3 * F), lambda t, *_: (0, 0)),
                pl.BlockSpec((F, 2 * F), lambda t, *_: (0, 0)),
                pl.BlockSpec((1, 2 * F), lambda t, *_: (0, 0)),
                pl.BlockSpec((2 * F, 2 * F), lambda t, *_: (0, 0)),
                pl.BlockSpec((1, 2 * F), lambda t, *_: (0, 0)),
                pl.BlockSpec((2 * F, F), lambda t, *_: (0, 0)),
                pl.BlockSpec((1, F), lambda t, *_: (0, 0)),
            ],
            out_specs=pl.BlockSpec((NA, 1), lambda t, *_: (0, 0)),
            scratch_shapes=[pltpu.VMEM((NA, F), jnp.float32)],
        ),
        out_shape=jax.ShapeDtypeStruct((NA, 1), jnp.float32),
    )(cap, mi, ps, wih_t, whh_t, bih, bhh, w1_t, b1, w2_t, b2, wo_t, bo)


def _selu(x):
    alpha = 1.6732632423543772848170429916717
    scale = 1.0507009873554804934193349852946
    return scale * jnp.where(x > 0, x, alpha * (jnp.exp(x) - 1.0))


def _readout_body(ps_ref, w1_ref, b1_ref, w2_ref, b2_ref, wo_ref, bo_ref,
                  out_ref):
    h = _selu(jax.lax.dot_general(ps_ref[...], w1_ref[...],
                                  (((1,), (0,)), ((), ())),
                                  preferred_element_type=jnp.float32)
              + b1_ref[...])
    h = _selu(jax.lax.dot_general(h, w2_ref[...], (((1,), (0,)), ((), ())),
                                  preferred_element_type=jnp.float32)
              + b2_ref[...])
    out_ref[...] = jax.lax.dot_general(h, wo_ref[...],
                                       (((1,), (0,)), ((), ())),
                                       preferred_element_type=jnp.float32) \
        + bo_ref[...]


def _readout(ps, w1_t, b1, w2_t, b2, wo_t, bo):
    return pl.pallas_call(
        _readout_body,
        out_shape=jax.ShapeDtypeStruct((NA, F), jnp.float32),
    )(ps, w1_t, b1, w2_t, b2, wo_t, bo)


def _pad_body(ls_ref, out_ref):
    nl = ls_ref.shape[0]
    out_ref[:nl, :] = ls_ref[...]
    out_ref[nl:, :] = jnp.zeros_like(out_ref[nl:, :])


def _pad_links(ls):
    return pl.pallas_call(
        _pad_body,
        out_shape=jax.ShapeDtypeStruct((NLp, F), jnp.float32),
    )(ls)


def kernel(link_state, path_state, link_id, path_id, sequence, num_actions,
           W_ih_m, W_hh_m, b_ih_m, b_hh_m, W_ih_u, W_hh_u, b_ih_u, b_hh_u,
           W_r1, b_r1, W_r2, b_r2, W_out, b_out):
    T = 4

    lid = link_id.astype(jnp.int32)
    pid = path_id.astype(jnp.int32)
    seq = sequence.astype(jnp.int32)

    wm_ih = W_ih_m.T
    wm_hh = W_hh_m.T
    wu_ih = W_ih_u.T
    wu_hh = W_hh_u.T
    bm_ih = b_ih_m.reshape(1, -1)
    bm_hh = b_hh_m.reshape(1, -1)
    bu_ih = b_ih_u.reshape(1, -1)
    bu_hh = b_hh_u.reshape(1, -1)
    w1_t = W_r1.T
    w2_t = W_r2.T
    wo_t = jnp.pad(W_out.T, ((0, 0), (0, F - W_out.shape[0])))
    b1 = b_r1.reshape(1, -1)
    b2 = b_r2.reshape(1, -1)
    bo = jnp.pad(b_out.reshape(1, -1), ((0, 0), (0, F - b_out.shape[0])))

    ls_ext = _pad_links(link_state)
    src, flat, capm, mi1 = _sc_invert(pid, seq, lid, ls_ext)
    src2 = src.reshape(S // 128, 128)
    flat2 = flat.reshape(E // 128, 128)
    lid2 = lid.reshape(E // 128, 128)
    cap = jnp.minimum(jnp.max(capm) + 1, num_actions).astype(jnp.int32)
    cap_arr = cap.reshape(1)
    nl_arr = jnp.full((1,), NL, jnp.int32)
    zeros_hbm = jnp.zeros((NLp, F), jnp.float32)

    ps = path_state
    for r in range(T - 1):
        mi = (mi1 if r == 0 else _sc_gather(ls_ext, src2)).reshape(ML, NA, F)
        m_seq, ps = _msg_scan(mi, ps, wm_ih, wm_hh, bm_ih, bm_hh, cap_arr)
        parts = _sc_agg(m_seq.reshape(S, F), flat2, lid2, zeros_hbm)
        ls_ext = _link_update(parts, ls_ext, wu_ih, wu_hh, bu_ih, bu_hh,
                              nl_arr)
    mi = _sc_gather(ls_ext, src2).reshape(ML, NA, F)
    return _msg_scan_readout(mi, ps, wm_ih, wm_hh, bm_ih, bm_hh, cap_arr,
                             w1_t, b1, w2_t, b2, wo_t, bo)


# speedup vs baseline: 1.1487x; 1.0516x over previous
import functools

import jax
import jax.numpy as jnp
from jax import lax
from jax.experimental import pallas as pl
from jax.experimental.pallas import tpu as pltpu
from jax.experimental.pallas import tpu_sc as plsc

ML = 8
F = 128
PAD = 112

NL = 10000
NA = 4096
E = 32768
S = ML * NA
NLp = NL + PAD
NC = 2
NS = 16
NW = NC * NS
EPW = E // NW
SPW = S // NW
RPS = NLp // NS

_SC_MESH = plsc.VectorSubcoreMesh(core_axis_name="c", subcore_axis_name="s")
_SC_PARAMS = pltpu.CompilerParams(needs_layout_passes=False)


def _ds(base, size):
    return pl.ds(pl.multiple_of(base, 8), size)


def _inv_body(pid_hbm, seq_hbm, lid_hbm, ls_hbm, src_hbm, flat_hbm, cap_hbm,
              mi_hbm, pid_v, seq_v, lid_v, src_v, flat_v, cap_v,
              rows_a, rows_b, sg_a, sg_b, sw_a, sw_b):
    w = lax.axis_index("s") * NC + lax.axis_index("c")
    lo = w * SPW
    pltpu.sync_copy(pid_hbm, pid_v)
    pltpu.sync_copy(seq_hbm, seq_v)
    pltpu.sync_copy(lid_hbm, lid_v)

    def init_body(j, carry):
        spread = (lax.iota(jnp.int32, 16) + j * 16) & 63
        src_v[pl.ds(j * 16, 16)] = NL + spread
        return carry

    lax.fori_loop(0, SPW // 16, init_body, 0)

    def scan_body(i, carry):
        for u in range(2):
            base = i * 32 + u * 16
            sq = seq_v[pl.ds(base, 16)]
            f = sq * NA + pid_v[pl.ds(base, 16)]
            l = lid_v[pl.ds(base, 16)]
            fl = f - lo
            m = (fl >= 0) & (fl < SPW)
            plsc.store_scatter(src_v, [fl], l, mask=m)
        return carry

    lax.fori_loop(0, E // 32, scan_body, 0)
    pltpu.sync_copy(src_v, src_hbm.at[_ds(lo, SPW)])

    ebase = w * EPW

    def flat_body(k, mx):
        sq = seq_v[pl.ds(ebase + k * 16, 16)]
        f = sq * NA + pid_v[pl.ds(ebase + k * 16, 16)]
        flat_v[pl.ds(k * 16, 16)] = f
        return jnp.maximum(mx, sq)

    mx = lax.fori_loop(0, EPW // 16, flat_body, jnp.zeros((16,), jnp.int32))
    pltpu.sync_copy(flat_v, flat_hbm.at[_ds(ebase, EPW)])
    for r in range(8):
        for j in range(8):
            cap_v[r, pl.ds(j * 16, 16)] = mx
    pltpu.sync_copy(cap_v, cap_hbm.at[_ds(w * 8, 8)])
    rows = (rows_a, rows_b)
    sg = (sg_a, sg_b)
    sw = (sw_a, sw_b)
    gd = [None, None]
    wd = [None, None]
    gd[0] = pltpu.async_copy(ls_hbm.at[src_v.at[pl.ds(0, 64)]], rows_a,
                             sg_a)
    for cc in range(16):
        b = cc & 1
        gd[b].wait()
        if cc + 1 < 16:
            if cc >= 1:
                wd[1 - b].wait()
            gd[1 - b] = pltpu.async_copy(
                ls_hbm.at[src_v.at[pl.ds((cc + 1) * 64, 64)]],
                rows[1 - b], sg[1 - b])
        wd[b] = pltpu.async_copy(rows[b],
                                 mi_hbm.at[_ds(w * SPW + cc * 64, 64)],
                                 sw[b])
    wd[0].wait()
    wd[1].wait()


_sc_invert = functools.partial(
    pl.kernel, _inv_body, mesh=_SC_MESH, compiler_params=_SC_PARAMS,
    out_type=(jax.ShapeDtypeStruct((S,), jnp.int32),
              jax.ShapeDtypeStruct((E,), jnp.int32),
              jax.ShapeDtypeStruct((NW * 8, 128), jnp.int32),
              jax.ShapeDtypeStruct((S, F), jnp.float32)),
    scratch_types=[pltpu.VMEM((E,), jnp.int32),
                   pltpu.VMEM((E,), jnp.int32),
                   pltpu.VMEM((E,), jnp.int32),
                   pltpu.VMEM((SPW,), jnp.int32),
                   pltpu.VMEM((EPW,), jnp.int32),
                   pltpu.VMEM((8, 128), jnp.int32),
                   pltpu.VMEM((64, F), jnp.float32),
                   pltpu.VMEM((64, F), jnp.float32),
                   pltpu.SemaphoreType.DMA, pltpu.SemaphoreType.DMA,
                   pltpu.SemaphoreType.DMA, pltpu.SemaphoreType.DMA],
)()


def _gat_body(ls_hbm, src2_hbm, mi_hbm, idx_v, rows_a, rows_b,
              sg_a, sg_b, sw_a, sw_b):
    w = lax.axis_index("s") * NC + lax.axis_index("c")
    pltpu.sync_copy(src2_hbm.at[_ds(w * 8, 8)], idx_v)
    rows = (rows_a, rows_b)
    sg = (sg_a, sg_b)
    sw = (sw_a, sw_b)
    gd = [None, None]
    wd = [None, None]
    gd[0] = pltpu.async_copy(ls_hbm.at[idx_v.at[0]], rows_a, sg_a)
    for cc in range(8):
        b = cc & 1
        gd[b].wait()
        if cc + 1 < 8:
            if cc >= 1:
                wd[1 - b].wait()
            gd[1 - b] = pltpu.async_copy(ls_hbm.at[idx_v.at[cc + 1]],
                                         rows[1 - b], sg[1 - b])
        wd[b] = pltpu.async_copy(rows[b],
                                 mi_hbm.at[_ds(w * SPW + cc * 128, 128)],
                                 sw[b])
    wd[0].wait()
    wd[1].wait()


_sc_gather = functools.partial(
    pl.kernel, _gat_body, mesh=_SC_MESH, compiler_params=_SC_PARAMS,
    out_type=jax.ShapeDtypeStruct((S, F), jnp.float32),
    scratch_types=[pltpu.VMEM((8, 128), jnp.int32),
                   pltpu.VMEM((128, F), jnp.float32),
                   pltpu.VMEM((128, F), jnp.float32),
                   pltpu.SemaphoreType.DMA, pltpu.SemaphoreType.DMA,
                   pltpu.SemaphoreType.DMA, pltpu.SemaphoreType.DMA],
)()


def _agg_body(mseq_hbm, flat2_hbm, lid2_hbm, zeros_hbm, parts_hbm,
              fidx, lidx, rows_a, rows_b, sp, sg_a, sg_b, sw_a, sw_b):
    c = lax.axis_index("c")
    s = lax.axis_index("s")
    w = s * NC + c
    zd = pltpu.async_copy(zeros_hbm.at[_ds(s * RPS, RPS)],
                          sp.at[_ds(s * RPS, RPS)], sw_a)
    pltpu.sync_copy(flat2_hbm.at[_ds(w * 8, 8)], fidx)
    pltpu.sync_copy(lid2_hbm.at[_ds(w * 8, 8)], lidx)
    rows = (rows_a, rows_b)
    sg = (sg_a, sg_b)
    sw = (sw_a, sw_b)
    gd = [None, None]
    wd = [None, None]
    gd[0] = pltpu.async_copy(mseq_hbm.at[fidx.at[0]], rows_a, sg_a)
    gd[1] = pltpu.async_copy(mseq_hbm.at[fidx.at[1]], rows_b, sg_b)
    zd.wait()
    plsc.subcore_barrier()
    for cc in range(8):
        b = cc & 1
        gd[b].wait()
        wd[b] = pltpu.async_copy(rows[b], sp.at[lidx.at[cc]], sw[b],
                                 add=True)
        if cc + 2 < 8:
            wd[b].wait()
            gd[b] = pltpu.async_copy(mseq_hbm.at[fidx.at[cc + 2]],
                                     rows[b], sg[b])
    wd[6 & 1].wait()
    wd[7 & 1].wait()
    plsc.subcore_barrier()
    pltpu.sync_copy(sp.at[_ds(s * RPS, RPS)],
                    parts_hbm.at[c, _ds(s * RPS, RPS)])


_sc_agg = functools.partial(
    pl.kernel, _agg_body, mesh=_SC_MESH, compiler_params=_SC_PARAMS,
    out_type=jax.ShapeDtypeStruct((NC, NLp, F), jnp.float32),
    scratch_types=[pltpu.VMEM((8, 128), jnp.int32),
                   pltpu.VMEM((8, 128), jnp.int32),
                   pltpu.VMEM((128, F), jnp.float32),
                   pltpu.VMEM((128, F), jnp.float32),
                   pltpu.VMEM_SHARED((NLp, F), jnp.float32),
                   pltpu.SemaphoreType.DMA, pltpu.SemaphoreType.DMA,
                   pltpu.SemaphoreType.DMA, pltpu.SemaphoreType.DMA],
)()


def _gru_math(x, h, gi_w, gh_w, b_ih, b_hh):
    gi = jax.lax.dot_general(x, gi_w, (((1,), (0,)), ((), ())),
                             preferred_element_type=jnp.float32) + b_ih
    gh = jax.lax.dot_general(h, gh_w, (((1,), (0,)), ((), ())),
                             preferred_element_type=jnp.float32) + b_hh
    r = jax.nn.sigmoid(gi[:, :F] + gh[:, :F])
    z = jax.nn.sigmoid(gi[:, F:2 * F] + gh[:, F:2 * F])
    n = jnp.tanh(gi[:, 2 * F:] + r * gh[:, 2 * F:])
    return (1.0 - z) * n + z * h


def _scan_body(cap_ref, mi_ref, ps_ref, wih_ref, whh_ref, bih_ref, bhh_ref,
               out_ref, psout_ref, h_ref):
    t = pl.program_id(0)

    @pl.when(t == 0)
    def _():
        h_ref[...] = ps_ref[...]

    x = mi_ref[0]
    h = h_ref[...]
    h_new = _gru_math(x, h, wih_ref[...], whh_ref[...], bih_ref[...],
                      bhh_ref[...])
    keep = t < cap_ref[0]
    h2 = jnp.where(keep, h_new, h)
    h_ref[...] = h2
    out_ref[0] = h2

    @pl.when(t == ML - 1)
    def _():
        psout_ref[...] = h2


def _msg_scan(mi, ps, wih_t, whh_t, bih, bhh, cap):
    return pl.pallas_call(
        _scan_body,
        grid_spec=pltpu.PrefetchScalarGridSpec(
            num_scalar_prefetch=1,
            grid=(ML,),
            in_specs=[
                pl.BlockSpec((1, NA, F), lambda t, *_: (t, 0, 0)),
                pl.BlockSpec((NA, F), lambda t, *_: (0, 0)),
                pl.BlockSpec((F, 3 * F), lambda t, *_: (0, 0)),
                pl.BlockSpec((F, 3 * F), lambda t, *_: (0, 0)),
                pl.BlockSpec((1, 3 * F), lambda t, *_: (0, 0)),
                pl.BlockSpec((1, 3 * F), lambda t, *_: (0, 0)),
            ],
            out_specs=[pl.BlockSpec((1, NA, F), lambda t, *_: (t, 0, 0)),
                       pl.BlockSpec((NA, F), lambda t, *_: (0, 0))],
            scratch_shapes=[pltpu.VMEM((NA, F), jnp.float32)],
        ),
        out_shape=[jax.ShapeDtypeStruct((ML, NA, F), jnp.float32),
                   jax.ShapeDtypeStruct((NA, F), jnp.float32)],
    )(cap, mi, ps, wih_t, whh_t, bih, bhh)


def _update_body(nl_ref, x_ref, h_ref, wih_ref, whh_ref, bih_ref, bhh_ref,
                 out_ref):
    i = pl.program_id(0)
    rows = h_ref.shape[0]
    x = x_ref[0] + x_ref[1]
    h = h_ref[...]
    h_new = _gru_math(x, h, wih_ref[...], whh_ref[...], bih_ref[...],
                      bhh_ref[...])
    row = jax.lax.broadcasted_iota(jnp.int32, (rows, 1), 0) + i * rows
    out_ref[...] = jnp.where(row < nl_ref[0], h_new, 0.0)


def _link_update(parts, ls_ext, wih_t, whh_t, bih, bhh, nl):
    blk = NLp // 4
    return pl.pallas_call(
        _update_body,
        grid_spec=pltpu.PrefetchScalarGridSpec(
            num_scalar_prefetch=1,
            grid=(4,),
            in_specs=[
                pl.BlockSpec((NC, blk, F), lambda i, *_: (0, i, 0)),
                pl.BlockSpec((blk, F), lambda i, *_: (i, 0)),
                pl.BlockSpec((F, 3 * F), lambda i, *_: (0, 0)),
                pl.BlockSpec((F, 3 * F), lambda i, *_: (0, 0)),
                pl.BlockSpec((1, 3 * F), lambda i, *_: (0, 0)),
                pl.BlockSpec((1, 3 * F), lambda i, *_: (0, 0)),
            ],
            out_specs=pl.BlockSpec((blk, F), lambda i, *_: (i, 0)),
            scratch_shapes=[],
        ),
        out_shape=jax.ShapeDtypeStruct((NLp, F), jnp.float32),
    )(nl, parts, ls_ext, wih_t, whh_t, bih, bhh)


def _scan_ro_body(cap_ref, mi_ref, ps_ref, wih_ref, whh_ref, bih_ref,
                  bhh_ref, w1_ref, b1_ref, w2_ref, b2_ref, wo_ref, bo_ref,
                  out_ref, h_ref):
    t = pl.program_id(0)

    @pl.when(t == 0)
    def _():
        h_ref[...] = ps_ref[...]

    x = mi_ref[0]
    h = h_ref[...]
    h_new = _gru_math(x, h, wih_ref[...], whh_ref[...], bih_ref[...],
                      bhh_ref[...])
    keep = t < cap_ref[0]
    h2 = jnp.where(keep, h_new, h)
    h_ref[...] = h2

    @pl.when(t == ML - 1)
    def _():
        hh = _selu(jax.lax.dot_general(h2, w1_ref[...],
                                       (((1,), (0,)), ((), ())),
                                       preferred_element_type=jnp.float32)
                   + b1_ref[...])
        hh = _selu(jax.lax.dot_general(hh, w2_ref[...],
                                       (((1,), (0,)), ((), ())),
                                       preferred_element_type=jnp.float32)
                   + b2_ref[...])
        res = jax.lax.dot_general(hh, wo_ref[...],
                                  (((1,), (0,)), ((), ())),
                                  preferred_element_type=jnp.float32
                                  ) + bo_ref[...]
        out_ref[...] = res[:, :1]


def _msg_scan_readout(mi, ps, wih_t, whh_t, bih, bhh, cap,
                      w1_t, b1, w2_t, b2, wo_t, bo):
    return pl.pallas_call(
        _scan_ro_body,
        grid_spec=pltpu.PrefetchScalarGridSpec(
            num_scalar_prefetch=1,
            grid=(ML,),
            in_specs=[
                pl.BlockSpec((1, NA, F), lambda t, *_: (t, 0, 0)),
                pl.BlockSpec((NA, F), lambda t, *_: (0, 0)),
                pl.BlockSpec((F, 3 * F), lambda t, *_: (0, 0)),
                pl.BlockSpec((F, 3 * F), lambda t, *_: (0, 0)),
                pl.BlockSpec((1, 3 * F), lambda t, *_: (0, 0)),
                pl.BlockSpec((1, 3 * F), lambda t, *_: (0, 0)),
                pl.BlockSpec((F, 2 * F), lambda t, *_: (0, 0)),
                pl.BlockSpec((1, 2 * F), lambda t, *_: (0, 0)),
                pl.BlockSpec((2 * F, 2 * F), lambda t, *_: (0, 0)),
                pl.BlockSpec((1, 2 * F), lambda t, *_: (0, 0)),
                pl.BlockSpec((2 * F, F), lambda t, *_: (0, 0)),
                pl.BlockSpec((1, F), lambda t, *_: (0, 0)),
            ],
            out_specs=pl.BlockSpec((NA, 1), lambda t, *_: (0, 0)),
            scratch_shapes=[pltpu.VMEM((NA, F), jnp.float32)],
        ),
        out_shape=jax.ShapeDtypeStruct((NA, 1), jnp.float32),
    )(cap, mi, ps, wih_t, whh_t, bih, bhh, w1_t, b1, w2_t, b2, wo_t, bo)


def _selu(x):
    alpha = 1.6732632423543772848170429916717
    scale = 1.0507009873554804934193349852946
    return scale * jnp.where(x > 0, x, alpha * (jnp.exp(x) - 1.0))


def _readout_body(ps_ref, w1_ref, b1_ref, w2_ref, b2_ref, wo_ref, bo_ref,
                  out_ref):
    h = _selu(jax.lax.dot_general(ps_ref[...], w1_ref[...],
                                  (((1,), (0,)), ((), ())),
                                  preferred_element_type=jnp.float32)
              + b1_ref[...])
    h = _selu(jax.lax.dot_general(h, w2_ref[...], (((1,), (0,)), ((), ())),
                                  preferred_element_type=jnp.float32)
              + b2_ref[...])
    out_ref[...] = jax.lax.dot_general(h, wo_ref[...],
                                       (((1,), (0,)), ((), ())),
                                       preferred_element_type=jnp.float32) \
        + bo_ref[...]


def _readout(ps, w1_t, b1, w2_t, b2, wo_t, bo):
    return pl.pallas_call(
        _readout_body,
        out_shape=jax.ShapeDtypeStruct((NA, F), jnp.float32),
    )(ps, w1_t, b1, w2_t, b2, wo_t, bo)


def _pad_body(ls_ref, out_ref):
    nl = ls_ref.shape[0]
    out_ref[:nl, :] = ls_ref[...]
    out_ref[nl:, :] = jnp.zeros_like(out_ref[nl:, :])


def _pad_links(ls):
    return pl.pallas_call(
        _pad_body,
        out_shape=jax.ShapeDtypeStruct((NLp, F), jnp.float32),
    )(ls)


def kernel(link_state, path_state, link_id, path_id, sequence, num_actions,
           W_ih_m, W_hh_m, b_ih_m, b_hh_m, W_ih_u, W_hh_u, b_ih_u, b_hh_u,
           W_r1, b_r1, W_r2, b_r2, W_out, b_out):
    T = 4

    lid = link_id.astype(jnp.int32)
    pid = path_id.astype(jnp.int32)
    seq = sequence.astype(jnp.int32)

    wm_ih = W_ih_m.T
    wm_hh = W_hh_m.T
    wu_ih = W_ih_u.T
    wu_hh = W_hh_u.T
    bm_ih = b_ih_m.reshape(1, -1)
    bm_hh = b_hh_m.reshape(1, -1)
    bu_ih = b_ih_u.reshape(1, -1)
    bu_hh = b_hh_u.reshape(1, -1)
    w1_t = W_r1.T
    w2_t = W_r2.T
    wo_t = jnp.pad(W_out.T, ((0, 0), (0, F - W_out.shape[0])))
    b1 = b_r1.reshape(1, -1)
    b2 = b_r2.reshape(1, -1)
    bo = jnp.pad(b_out.reshape(1, -1), ((0, 0), (0, F - b_out.shape[0])))

    ls_ext = jnp.pad(link_state, ((0, PAD), (0, 0)))
    src, flat, capm, mi1 = _sc_invert(pid, seq, lid, ls_ext)
    src2 = src.reshape(S // 128, 128)
    flat2 = flat.reshape(E // 128, 128)
    lid2 = lid.reshape(E // 128, 128)
    cap = jnp.minimum(jnp.max(capm) + 1, num_actions).astype(jnp.int32)
    cap_arr = cap.reshape(1)
    nl_arr = jnp.full((1,), NL, jnp.int32)
    zeros_hbm = jnp.zeros((NLp, F), jnp.float32)

    ps = path_state
    for r in range(T - 1):
        mi = (mi1 if r == 0 else _sc_gather(ls_ext, src2)).reshape(ML, NA, F)
        m_seq, ps = _msg_scan(mi, ps, wm_ih, wm_hh, bm_ih, bm_hh, cap_arr)
        parts = _sc_agg(m_seq.reshape(S, F), flat2, lid2, zeros_hbm)
        ls_ext = _link_update(parts, ls_ext, wu_ih, wu_hh, bu_ih, bu_hh,
                              nl_arr)
    mi = _sc_gather(ls_ext, src2).reshape(ML, NA, F)
    return _msg_scan_readout(mi, ps, wm_ih, wm_hh, bm_ih, bm_hh, cap_arr,
                             w1_t, b1, w2_t, b2, wo_t, bo)
